# Initial kernel scaffold; baseline (speedup 1.0000x reference)
import jax
import jax.numpy as jnp
from jax.experimental import pallas as pl


def kernel(x, edge_attr, Wq, bq, Wk, bk, We, be, Wv, bv, edge_index):
    raise NotImplementedError("write your pallas kernel here")


# speedup vs baseline: 15.5933x; 15.5933x over previous
import functools

import jax
import jax.numpy as jnp
import numpy as np
from jax import lax
from jax.experimental import pallas as pl
from jax.experimental.pallas import tpu as pltpu
from jax.experimental.pallas import tpu_sc as plsc

N_NODES = 10000
E_EDGES = 160000
D = 256
H = 8
DH = 32
SCALE = 1.0 / np.sqrt(DH)

NC = 2
NS = 16
NW = NC * NS

_PROJ_TILE = 1000


def _proj_body(x_ref, wq, bq, wk, bk, wv, bv, q_out, k_out, v_out):
    x = x_ref[...]
    q_out[...] = jnp.dot(x, wq[...], preferred_element_type=jnp.float32) + bq[...]
    k_out[...] = jnp.dot(x, wk[...], preferred_element_type=jnp.float32) + bk[...]
    v_out[...] = jnp.dot(x, wv[...], preferred_element_type=jnp.float32) + bv[...]


def _proj(x, wq, bq, wk, bk, wv, bv):
    grid = (N_NODES // _PROJ_TILE,)
    wspec = pl.BlockSpec((D, D), lambda i: (0, 0))
    bspec = pl.BlockSpec((1, D), lambda i: (0, 0))
    rspec = pl.BlockSpec((_PROJ_TILE, D), lambda i: (i, 0))
    return pl.pallas_call(
        _proj_body,
        grid=grid,
        in_specs=[rspec, wspec, bspec, wspec, bspec, wspec, bspec],
        out_specs=[rspec, rspec, rspec],
        out_shape=[jax.ShapeDtypeStruct((N_NODES, D), jnp.float32)] * 3,
    )(x, wq, bq, wk, bk, wv, bv)


_EPW = E_EDGES // NW
_GCH = 128
_GFULL = _EPW // _GCH
_GTAIL = _EPW - _GFULL * _GCH


def _gather_body(ktab, qtab, vtab, src, dst, ks_out, qd_out, vs_out,
                 idx_v, rows_v, idxt_v, rowst_v, sem):
    wid = lax.axis_index("s") * NC + lax.axis_index("c")
    base = wid * _EPW

    def do_chunk(off, idxb, rowsb, n):
        pltpu.sync_copy(src.at[pl.ds(off, n)], idxb)
        pltpu.async_copy(ktab.at[idxb], rowsb, sem).wait()
        pltpu.sync_copy(rowsb, ks_out.at[pl.ds(off, n)])
        pltpu.sync_copy(dst.at[pl.ds(off, n)], idxb)
        pltpu.async_copy(qtab.at[idxb], rowsb, sem).wait()
        pltpu.sync_copy(rowsb, qd_out.at[pl.ds(off, n)])
        pltpu.sync_copy(src.at[pl.ds(off, n)], idxb)
        pltpu.async_copy(vtab.at[idxb], rowsb, sem).wait()
        pltpu.sync_copy(rowsb, vs_out.at[pl.ds(off, n)])

    def body(i, carry):
        do_chunk(base + i * _GCH, idx_v, rows_v, _GCH)
        return carry

    lax.fori_loop(0, _GFULL, body, 0)
    do_chunk(base + _GFULL * _GCH, idxt_v, rowst_v, _GTAIL)


_gather = functools.partial(
    pl.kernel,
    mesh=plsc.VectorSubcoreMesh(core_axis_name="c", subcore_axis_name="s",
                                num_cores=NC, num_subcores=NS),
    out_type=[jax.ShapeDtypeStruct((E_EDGES, D), jnp.float32)] * 3,
    scratch_types=[
        pltpu.VMEM((_GCH,), jnp.int32),
        pltpu.VMEM((_GCH, D), jnp.float32),
        pltpu.VMEM((_GTAIL,), jnp.int32),
        pltpu.VMEM((_GTAIL, D), jnp.float32),
        pltpu.SemaphoreType.DMA,
    ],
)(_gather_body)


_ETILE = 640


def _edge_body(ea_ref, ks_ref, qd_ref, vs_ref, dst_ref, we_ref, be_ref,
               msg3_out, scp_out):
    eh = jnp.dot(ea_ref[...], we_ref[...], preferred_element_type=jnp.float32) + be_ref[...]
    pe = ks_ref[...] * qd_ref[...] * eh
    d_i = lax.broadcasted_iota(jnp.int32, (D, H), 0)
    h_i = lax.broadcasted_iota(jnp.int32, (D, H), 1)
    seg = jnp.where(d_i // DH == h_i, SCALE, 0.0)
    raw = jnp.dot(pe, seg, preferred_element_type=jnp.float32)
    sc = jnp.exp(jnp.clip(raw, -5.0, 5.0))
    d_i2 = lax.broadcasted_iota(jnp.int32, (H, D), 1)
    h_i2 = lax.broadcasted_iota(jnp.int32, (H, D), 0)
    rep = jnp.where(d_i2 // DH == h_i2, 1.0, 0.0)
    msg = vs_ref[...] * jnp.dot(sc, rep, preferred_element_type=jnp.float32)
    msg3_out[0] = msg[:, :D // 2]
    msg3_out[1] = msg[:, D // 2:]
    dd = dst_ref[...]
    k_i = lax.broadcasted_iota(jnp.int32, (_ETILE, 8), 1)
    oh = jnp.where(dd - (dd // 8) * 8 == k_i, 1.0, 0.0)
    k_i2 = lax.broadcasted_iota(jnp.int32, (8, _HD), 0)
    d_i3 = lax.broadcasted_iota(jnp.int32, (8, _HD), 1)
    r8 = jnp.where(d_i3 // (2 * H) == k_i2, 1.0, 0.0)
    ohrep = jnp.dot(oh, r8, preferred_element_type=jnp.float32)
    sc_pad = jnp.concatenate([sc, jnp.zeros_like(sc)], axis=1)
    sc_tile = jnp.concatenate([sc_pad] * 8, axis=1)
    scp_out[...] = sc_tile * ohrep


def _edge(edge_attr, ks, qd, vs, dst2d, we, be):
    grid = (E_EDGES // _ETILE,)
    rspec = pl.BlockSpec((_ETILE, D), lambda i: (i, 0))
    return pl.pallas_call(
        _edge_body,
        grid=grid,
        in_specs=[rspec, rspec, rspec, rspec,
                  pl.BlockSpec((_ETILE, 1), lambda i: (i, 0)),
                  pl.BlockSpec((D, D), lambda i: (0, 0)),
                  pl.BlockSpec((1, D), lambda i: (0, 0))],
        out_specs=[pl.BlockSpec((NC, _ETILE, D // 2), lambda i: (0, i, 0)),
                   pl.BlockSpec((_ETILE, _HD), lambda i: (i, 0))],
        out_shape=[jax.ShapeDtypeStruct((NC, E_EDGES, D // 2), jnp.float32),
                   jax.ShapeDtypeStruct((E_EDGES, _HD), jnp.float32)],
    )(edge_attr, ks, qd, vs, dst2d, we, be)


_EPS = E_EDGES // NS
_SCH = 128
_SFULL = _EPS // _SCH
_STAIL = _EPS - _SFULL * _SCH
_NPAD = 10240
_RPS = _NPAD // NS
_HD = D // 2
_ZW = 2 * H
_NZ = _NPAD // 8
_ZRPS = _NZ // NS


def _scatter_body(msg3, scp, dst, dst8, wv3_out, z_out,
                  acc, zacc, idx_v, idx2_v, dat_v, zdat_v,
                  idxt_v, idxt2_v, idxz_v, idxz2_v, sem):
    c = lax.axis_index("c")
    s = lax.axis_index("s")
    zero16 = jnp.zeros((16,), jnp.float32)

    def zbody(t, carry):
        r = t // 8
        j = t - r * 8
        dat_v[r, pl.ds(j * 16, 16)] = zero16
        return carry

    lax.fori_loop(0, _SCH * 8, zbody, 0)

    def set_block_indices(kk):
        base_r = (s * 5 + kk) * _SCH
        for j in range(8):
            idxz_v[pl.ds(j * 16, 16)] = base_r + j * 16 + lax.iota(jnp.int32, 16)

    def set_z_indices():
        base_r = s * _ZRPS
        for j in range(_ZRPS // 16):
            idxz2_v[pl.ds(j * 16, 16)] = base_r + j * 16 + lax.iota(jnp.int32, 16)

    for kk in range(5):
        set_block_indices(kk)
        pltpu.sync_copy(dat_v, acc.at[idxz_v])
    set_z_indices()
    pltpu.sync_copy(dat_v.at[pl.ds(0, _ZRPS)], zacc.at[idxz2_v])
    plsc.subcore_barrier()

    base = s * _EPS

    def do_chunk(off, idxb, idx2b, datb, zdatb, n):
        pltpu.sync_copy(dst.at[pl.ds(off, n)], idxb)
        pltpu.sync_copy(dst8.at[pl.ds(off, n)], idx2b)
        pltpu.sync_copy(msg3.at[c, pl.ds(off, n)], datb)
        pltpu.sync_copy(datb, acc.at[idxb], add=True)
        pltpu.sync_copy(scp.at[pl.ds(off, n)], zdatb)
        pltpu.sync_copy(zdatb, zacc.at[idx2b], add=True)

    def body(i, carry):
        do_chunk(base + i * _SCH, idx_v, idx2_v, dat_v, zdat_v, _SCH)
        return carry

    lax.fori_loop(0, _SFULL, body, 0)
    do_chunk(base + _SFULL * _SCH, idxt_v, idxt2_v,
             dat_v.at[pl.ds(0, _STAIL)], zdat_v.at[pl.ds(0, _STAIL)], _STAIL)
    plsc.subcore_barrier()

    for kk in range(5):
        set_block_indices(kk)
        row = (s * 5 + kk) * _SCH
        pltpu.async_copy(acc.at[idxz_v], dat_v, sem).wait()
        pltpu.sync_copy(dat_v, wv3_out.at[c, pl.ds(row, _SCH)])
    set_z_indices()
    pltpu.async_copy(zacc.at[idxz2_v], dat_v.at[pl.ds(0, _ZRPS)], sem).wait()
    pltpu.sync_copy(dat_v.at[pl.ds(0, _ZRPS)], z_out.at[pl.ds(s * _ZRPS, _ZRPS)])


_scatter = functools.partial(
    pl.kernel,
    mesh=plsc.VectorSubcoreMesh(core_axis_name="c", subcore_axis_name="s",
                                num_cores=NC, num_subcores=NS),
    out_type=[jax.ShapeDtypeStruct((NC, _NPAD, _HD), jnp.float32),
              jax.ShapeDtypeStruct((_NZ, _HD), jnp.float32)],
    scratch_types=[
        pltpu.VMEM_SHARED((_NPAD, _HD), jnp.float32),
        pltpu.VMEM_SHARED((_NZ, _HD), jnp.float32),
        pltpu.VMEM((_SCH,), jnp.int32),
        pltpu.VMEM((_SCH,), jnp.int32),
        pltpu.VMEM((_SCH, _HD), jnp.float32),
        pltpu.VMEM((_SCH, _HD), jnp.float32),
        pltpu.VMEM((_STAIL,), jnp.int32),
        pltpu.VMEM((_STAIL,), jnp.int32),
        pltpu.VMEM((_SCH,), jnp.int32),
        pltpu.VMEM((_ZRPS,), jnp.int32),
        pltpu.SemaphoreType.DMA,
    ],
)(_scatter_body)


_FTILE = 1000


def _fin_body(wva_ref, wvb_ref, z_ref, out_ref):
    h_i = lax.broadcasted_iota(jnp.int32, (_ZW, _HD), 0)
    d_i = lax.broadcasted_iota(jnp.int32, (_ZW, _HD), 1)
    rep_a = jnp.where(h_i == d_i // DH, 1.0, 0.0)
    rep_b = jnp.where(h_i == d_i // DH + H // 2, 1.0, 0.0)
    za = jnp.dot(z_ref[...], rep_a, preferred_element_type=jnp.float32)
    zb = jnp.dot(z_ref[...], rep_b, preferred_element_type=jnp.float32)
    out_ref[:, :_HD] = wva_ref[...] / (za + 1e-6)
    out_ref[:, _HD:] = wvb_ref[...] / (zb + 1e-6)


def _fin(wva, wvb, z):
    grid = (N_NODES // _FTILE,)
    return pl.pallas_call(
        _fin_body,
        grid=grid,
        in_specs=[pl.BlockSpec((_FTILE, _HD), lambda i: (i, 0)),
                  pl.BlockSpec((_FTILE, _HD), lambda i: (i, 0)),
                  pl.BlockSpec((_FTILE, _ZW), lambda i: (i, 0))],
        out_specs=pl.BlockSpec((_FTILE, D), lambda i: (i, 0)),
        out_shape=jax.ShapeDtypeStruct((N_NODES, D), jnp.float32),
    )(wva, wvb, z)


def kernel(x, edge_attr, Wq, bq, Wk, bk, We, be, Wv, bv, edge_index):
    src = edge_index[0]
    dst = edge_index[1]
    q, k, v = _proj(x, Wq, bq.reshape(1, D), Wk, bk.reshape(1, D),
                    Wv, bv.reshape(1, D))
    ks, qd, vs = _gather(k, q, v, src, dst)
    msg3, scp = _edge(edge_attr, ks, qd, vs, dst.reshape(E_EDGES, 1),
                      We, be.reshape(1, D))
    wv3, z2 = _scatter(msg3, scp, dst, dst // 8)
    return _fin(wv3[0], wv3[1], z2.reshape(_NPAD, _ZW))


# speedup vs baseline: 16.5768x; 1.0631x over previous
import functools

import jax
import jax.numpy as jnp
import numpy as np
from jax import lax
from jax.experimental import pallas as pl
from jax.experimental.pallas import tpu as pltpu
from jax.experimental.pallas import tpu_sc as plsc

N_NODES = 10000
E_EDGES = 160000
D = 256
H = 8
DH = 32
SCALE = 1.0 / np.sqrt(DH)

NC = 2
NS = 16
NW = NC * NS

_PROJ_TILE = 1000


def _proj_body(x_ref, wq, bq, wk, bk, wv, bv, q_out, k_out, v_out):
    x = x_ref[...]
    q_out[...] = jnp.dot(x, wq[...], preferred_element_type=jnp.float32) + bq[...]
    k_out[...] = jnp.dot(x, wk[...], preferred_element_type=jnp.float32) + bk[...]
    v_out[...] = jnp.dot(x, wv[...], preferred_element_type=jnp.float32) + bv[...]


def _proj(x, wq, bq, wk, bk, wv, bv):
    grid = (N_NODES // _PROJ_TILE,)
    wspec = pl.BlockSpec((D, D), lambda i: (0, 0))
    bspec = pl.BlockSpec((1, D), lambda i: (0, 0))
    rspec = pl.BlockSpec((_PROJ_TILE, D), lambda i: (i, 0))
    return pl.pallas_call(
        _proj_body,
        grid=grid,
        in_specs=[rspec, wspec, bspec, wspec, bspec, wspec, bspec],
        out_specs=[rspec, rspec, rspec],
        out_shape=[jax.ShapeDtypeStruct((N_NODES, D), jnp.float32)] * 3,
    )(x, wq, bq, wk, bk, wv, bv)


_EPW = E_EDGES // NW
_GCH = 128
_GFULL = _EPW // _GCH
_GTAIL = _EPW - _GFULL * _GCH


def _gather_body(ktab, qtab, vtab, src, dst, ks_out, qd_out, vs_out,
                 idxs_v, idxd_v, bufk_v, bufq_v, bufv_v,
                 idxst_v, idxdt_v, bufkt_v, bufqt_v, bufvt_v, gsem, wsem):
    wid = lax.axis_index("s") * NC + lax.axis_index("c")
    base = wid * _EPW

    def do_chunk(off, n, isv, idv, bk, bq, bv):
        pltpu.sync_copy(src.at[pl.ds(off, n)], isv)
        pltpu.sync_copy(dst.at[pl.ds(off, n)], idv)
        ck = pltpu.async_copy(ktab.at[isv], bk, gsem)
        cq = pltpu.async_copy(qtab.at[idv], bq, gsem)
        cv = pltpu.async_copy(vtab.at[isv], bv, gsem)
        ck.wait()
        cq.wait()
        cv.wait()
        wk = pltpu.async_copy(bk, ks_out.at[pl.ds(off, n)], wsem)
        wq = pltpu.async_copy(bq, qd_out.at[pl.ds(off, n)], wsem)
        wv = pltpu.async_copy(bv, vs_out.at[pl.ds(off, n)], wsem)
        wk.wait()
        wq.wait()
        wv.wait()

    def body(i, carry):
        do_chunk(base + i * _GCH, _GCH, idxs_v, idxd_v, bufk_v, bufq_v, bufv_v)
        return carry

    lax.fori_loop(0, _GFULL, body, 0)
    do_chunk(base + _GFULL * _GCH, _GTAIL, idxst_v, idxdt_v,
             bufkt_v, bufqt_v, bufvt_v)


_gather = functools.partial(
    pl.kernel,
    mesh=plsc.VectorSubcoreMesh(core_axis_name="c", subcore_axis_name="s",
                                num_cores=NC, num_subcores=NS),
    out_type=[jax.ShapeDtypeStruct((E_EDGES, D), jnp.float32)] * 3,
    scratch_types=[
        pltpu.VMEM((_GCH,), jnp.int32),
        pltpu.VMEM((_GCH,), jnp.int32),
        pltpu.VMEM((_GCH, D), jnp.float32),
        pltpu.VMEM((_GCH, D), jnp.float32),
        pltpu.VMEM((_GCH, D), jnp.float32),
        pltpu.VMEM((_GTAIL,), jnp.int32),
        pltpu.VMEM((_GTAIL,), jnp.int32),
        pltpu.VMEM((_GTAIL, D), jnp.float32),
        pltpu.VMEM((_GTAIL, D), jnp.float32),
        pltpu.VMEM((_GTAIL, D), jnp.float32),
        pltpu.SemaphoreType.DMA,
        pltpu.SemaphoreType.DMA,
    ],
)(_gather_body)


_ETILE = 640


def _edge_body(ea_ref, ks_ref, qd_ref, vs_ref, dst_ref, we_ref, be_ref,
               msg3_out, scp_out):
    eh = jnp.dot(ea_ref[...], we_ref[...], preferred_element_type=jnp.float32) + be_ref[...]
    pe = ks_ref[...] * qd_ref[...] * eh
    d_i = lax.broadcasted_iota(jnp.int32, (D, H), 0)
    h_i = lax.broadcasted_iota(jnp.int32, (D, H), 1)
    seg = jnp.where(d_i // DH == h_i, SCALE, 0.0)
    raw = jnp.dot(pe, seg, preferred_element_type=jnp.float32)
    sc = jnp.exp(jnp.clip(raw, -5.0, 5.0))
    d_i2 = lax.broadcasted_iota(jnp.int32, (H, D), 1)
    h_i2 = lax.broadcasted_iota(jnp.int32, (H, D), 0)
    rep = jnp.where(d_i2 // DH == h_i2, 1.0, 0.0)
    msg = vs_ref[...] * jnp.dot(sc, rep, preferred_element_type=jnp.float32)
    msg3_out[0] = msg[:, :D // 2]
    msg3_out[1] = msg[:, D // 2:]
    dd = dst_ref[...]
    k_i = lax.broadcasted_iota(jnp.int32, (_ETILE, 8), 1)
    oh = jnp.where(dd - (dd // 8) * 8 == k_i, 1.0, 0.0)
    k_i2 = lax.broadcasted_iota(jnp.int32, (8, _HD), 0)
    d_i3 = lax.broadcasted_iota(jnp.int32, (8, _HD), 1)
    r8 = jnp.where(d_i3 // (2 * H) == k_i2, 1.0, 0.0)
    ohrep = jnp.dot(oh, r8, preferred_element_type=jnp.float32)
    sc_pad = jnp.concatenate([sc, jnp.zeros_like(sc)], axis=1)
    sc_tile = jnp.concatenate([sc_pad] * 8, axis=1)
    scp_out[...] = sc_tile * ohrep


def _edge(edge_attr, ks, qd, vs, dst2d, we, be):
    grid = (E_EDGES // _ETILE,)
    rspec = pl.BlockSpec((_ETILE, D), lambda i: (i, 0))
    return pl.pallas_call(
        _edge_body,
        grid=grid,
        in_specs=[rspec, rspec, rspec, rspec,
                  pl.BlockSpec((_ETILE, 1), lambda i: (i, 0)),
                  pl.BlockSpec((D, D), lambda i: (0, 0)),
                  pl.BlockSpec((1, D), lambda i: (0, 0))],
        out_specs=[pl.BlockSpec((NC, _ETILE, D // 2), lambda i: (0, i, 0)),
                   pl.BlockSpec((_ETILE, _HD), lambda i: (i, 0))],
        out_shape=[jax.ShapeDtypeStruct((NC, E_EDGES, D // 2), jnp.float32),
                   jax.ShapeDtypeStruct((E_EDGES, _HD), jnp.float32)],
    )(edge_attr, ks, qd, vs, dst2d, we, be)


_EPS = E_EDGES // NS
_SCH = 128
_SFULL = _EPS // _SCH
_STAIL = _EPS - _SFULL * _SCH
_NPAD = 10240
_RPS = _NPAD // NS
_HD = D // 2
_ZW = 2 * H
_NZ = _NPAD // 8
_ZRPS = _NZ // NS


def _scatter_body(msg3, scp, dst, dst8, wv3_out, z_out,
                  acc, zacc, idx_v, idx2_v, dat_v, zdat_v,
                  idxt_v, idxt2_v, idxz_v, idxz2_v, sem):
    c = lax.axis_index("c")
    s = lax.axis_index("s")
    zero16 = jnp.zeros((16,), jnp.float32)

    def zbody(t, carry):
        r = t // 8
        j = t - r * 8
        dat_v[r, pl.ds(j * 16, 16)] = zero16
        return carry

    lax.fori_loop(0, _SCH * 8, zbody, 0)

    def set_block_indices(kk):
        base_r = (s * 5 + kk) * _SCH
        for j in range(8):
            idxz_v[pl.ds(j * 16, 16)] = base_r + j * 16 + lax.iota(jnp.int32, 16)

    def set_z_indices():
        base_r = s * _ZRPS
        for j in range(_ZRPS // 16):
            idxz2_v[pl.ds(j * 16, 16)] = base_r + j * 16 + lax.iota(jnp.int32, 16)

    for kk in range(5):
        set_block_indices(kk)
        pltpu.sync_copy(dat_v, acc.at[idxz_v])
    set_z_indices()
    pltpu.sync_copy(dat_v.at[pl.ds(0, _ZRPS)], zacc.at[idxz2_v])
    plsc.subcore_barrier()

    base = s * _EPS

    def do_chunk(off, idxb, idx2b, datb, zdatb, n):
        pltpu.sync_copy(dst.at[pl.ds(off, n)], idxb)
        pltpu.sync_copy(dst8.at[pl.ds(off, n)], idx2b)
        pltpu.sync_copy(msg3.at[c, pl.ds(off, n)], datb)
        pltpu.sync_copy(datb, acc.at[idxb], add=True)
        pltpu.sync_copy(scp.at[pl.ds(off, n)], zdatb)
        pltpu.sync_copy(zdatb, zacc.at[idx2b], add=True)

    def body(i, carry):
        do_chunk(base + i * _SCH, idx_v, idx2_v, dat_v, zdat_v, _SCH)
        return carry

    lax.fori_loop(0, _SFULL, body, 0)
    do_chunk(base + _SFULL * _SCH, idxt_v, idxt2_v,
             dat_v.at[pl.ds(0, _STAIL)], zdat_v.at[pl.ds(0, _STAIL)], _STAIL)
    plsc.subcore_barrier()

    for kk in range(5):
        set_block_indices(kk)
        row = (s * 5 + kk) * _SCH
        pltpu.async_copy(acc.at[idxz_v], dat_v, sem).wait()
        pltpu.sync_copy(dat_v, wv3_out.at[c, pl.ds(row, _SCH)])
    set_z_indices()
    pltpu.async_copy(zacc.at[idxz2_v], dat_v.at[pl.ds(0, _ZRPS)], sem).wait()
    pltpu.sync_copy(dat_v.at[pl.ds(0, _ZRPS)], z_out.at[pl.ds(s * _ZRPS, _ZRPS)])


_scatter = functools.partial(
    pl.kernel,
    mesh=plsc.VectorSubcoreMesh(core_axis_name="c", subcore_axis_name="s",
                                num_cores=NC, num_subcores=NS),
    out_type=[jax.ShapeDtypeStruct((NC, _NPAD, _HD), jnp.float32),
              jax.ShapeDtypeStruct((_NZ, _HD), jnp.float32)],
    scratch_types=[
        pltpu.VMEM_SHARED((_NPAD, _HD), jnp.float32),
        pltpu.VMEM_SHARED((_NZ, _HD), jnp.float32),
        pltpu.VMEM((_SCH,), jnp.int32),
        pltpu.VMEM((_SCH,), jnp.int32),
        pltpu.VMEM((_SCH, _HD), jnp.float32),
        pltpu.VMEM((_SCH, _HD), jnp.float32),
        pltpu.VMEM((_STAIL,), jnp.int32),
        pltpu.VMEM((_STAIL,), jnp.int32),
        pltpu.VMEM((_SCH,), jnp.int32),
        pltpu.VMEM((_ZRPS,), jnp.int32),
        pltpu.SemaphoreType.DMA,
    ],
)(_scatter_body)


_FTILE = 1000


def _fin_body(wva_ref, wvb_ref, z_ref, out_ref):
    h_i = lax.broadcasted_iota(jnp.int32, (_ZW, _HD), 0)
    d_i = lax.broadcasted_iota(jnp.int32, (_ZW, _HD), 1)
    rep_a = jnp.where(h_i == d_i // DH, 1.0, 0.0)
    rep_b = jnp.where(h_i == d_i // DH + H // 2, 1.0, 0.0)
    za = jnp.dot(z_ref[...], rep_a, preferred_element_type=jnp.float32)
    zb = jnp.dot(z_ref[...], rep_b, preferred_element_type=jnp.float32)
    out_ref[:, :_HD] = wva_ref[...] / (za + 1e-6)
    out_ref[:, _HD:] = wvb_ref[...] / (zb + 1e-6)


def _fin(wva, wvb, z):
    grid = (N_NODES // _FTILE,)
    return pl.pallas_call(
        _fin_body,
        grid=grid,
        in_specs=[pl.BlockSpec((_FTILE, _HD), lambda i: (i, 0)),
                  pl.BlockSpec((_FTILE, _HD), lambda i: (i, 0)),
                  pl.BlockSpec((_FTILE, _ZW), lambda i: (i, 0))],
        out_specs=pl.BlockSpec((_FTILE, D), lambda i: (i, 0)),
        out_shape=jax.ShapeDtypeStruct((N_NODES, D), jnp.float32),
    )(wva, wvb, z)


def kernel(x, edge_attr, Wq, bq, Wk, bk, We, be, Wv, bv, edge_index):
    src = edge_index[0]
    dst = edge_index[1]
    q, k, v = _proj(x, Wq, bq.reshape(1, D), Wk, bk.reshape(1, D),
                    Wv, bv.reshape(1, D))
    ks, qd, vs = _gather(k, q, v, src, dst)
    msg3, scp = _edge(edge_attr, ks, qd, vs, dst.reshape(E_EDGES, 1),
                      We, be.reshape(1, D))
    wv3, z2 = _scatter(msg3, scp, dst, dst // 8)
    return _fin(wv3[0], wv3[1], z2.reshape(_NPAD, _ZW))


# speedup vs baseline: 18.2802x; 1.1028x over previous
import functools

import jax
import jax.numpy as jnp
import numpy as np
from jax import lax
from jax.experimental import pallas as pl
from jax.experimental.pallas import tpu as pltpu
from jax.experimental.pallas import tpu_sc as plsc

N_NODES = 10000
E_EDGES = 160000
D = 256
H = 8
DH = 32
SCALE = 1.0 / np.sqrt(DH)

NC = 2
NS = 16
NW = NC * NS

_PROJ_TILE = 1000


def _proj_body(x_ref, wq, bq, wk, bk, wv, bv, q_out, k_out, v_out):
    x = x_ref[...]
    q_out[...] = jnp.dot(x, wq[...], preferred_element_type=jnp.float32) + bq[...]
    k_out[...] = jnp.dot(x, wk[...], preferred_element_type=jnp.float32) + bk[...]
    v_out[...] = jnp.dot(x, wv[...], preferred_element_type=jnp.float32) + bv[...]


def _proj(x, wq, bq, wk, bk, wv, bv):
    grid = (N_NODES // _PROJ_TILE,)
    wspec = pl.BlockSpec((D, D), lambda i: (0, 0))
    bspec = pl.BlockSpec((1, D), lambda i: (0, 0))
    rspec = pl.BlockSpec((_PROJ_TILE, D), lambda i: (i, 0))
    return pl.pallas_call(
        _proj_body,
        grid=grid,
        in_specs=[rspec, wspec, bspec, wspec, bspec, wspec, bspec],
        out_specs=[rspec, rspec, rspec],
        out_shape=[jax.ShapeDtypeStruct((N_NODES, D), jnp.float32)] * 3,
    )(x, wq, bq, wk, bk, wv, bv)


_EPW = E_EDGES // NW
_GCH = 128
_GFULL = _EPW // _GCH
_GTAIL = _EPW - _GFULL * _GCH


def _gather_body(ktab, qtab, vtab, src, dst, ks_out, qd_out, vs_out,
                 idxs_v, idxd_v, bufk_v, bufq_v, bufv_v,
                 idxst_v, idxdt_v, bufkt_v, bufqt_v, bufvt_v, gsem, wsem):
    wid = lax.axis_index("s") * NC + lax.axis_index("c")
    base = wid * _EPW

    def do_chunk(off, n, isv, idv, bk, bq, bv):
        pltpu.sync_copy(src.at[pl.ds(off, n)], isv)
        pltpu.sync_copy(dst.at[pl.ds(off, n)], idv)
        ck = pltpu.async_copy(ktab.at[isv], bk, gsem)
        cq = pltpu.async_copy(qtab.at[idv], bq, gsem)
        cv = pltpu.async_copy(vtab.at[isv], bv, gsem)
        ck.wait()
        cq.wait()
        cv.wait()
        wk = pltpu.async_copy(bk, ks_out.at[pl.ds(off, n)], wsem)
        wq = pltpu.async_copy(bq, qd_out.at[pl.ds(off, n)], wsem)
        wv = pltpu.async_copy(bv, vs_out.at[pl.ds(off, n)], wsem)
        wk.wait()
        wq.wait()
        wv.wait()

    def body(i, carry):
        do_chunk(base + i * _GCH, _GCH, idxs_v, idxd_v, bufk_v, bufq_v, bufv_v)
        return carry

    lax.fori_loop(0, _GFULL, body, 0)
    do_chunk(base + _GFULL * _GCH, _GTAIL, idxst_v, idxdt_v,
             bufkt_v, bufqt_v, bufvt_v)


_gather = functools.partial(
    pl.kernel,
    mesh=plsc.VectorSubcoreMesh(core_axis_name="c", subcore_axis_name="s",
                                num_cores=NC, num_subcores=NS),
    out_type=[jax.ShapeDtypeStruct((E_EDGES, D), jnp.float32)] * 3,
    scratch_types=[
        pltpu.VMEM((_GCH,), jnp.int32),
        pltpu.VMEM((_GCH,), jnp.int32),
        pltpu.VMEM((_GCH, D), jnp.float32),
        pltpu.VMEM((_GCH, D), jnp.float32),
        pltpu.VMEM((_GCH, D), jnp.float32),
        pltpu.VMEM((_GTAIL,), jnp.int32),
        pltpu.VMEM((_GTAIL,), jnp.int32),
        pltpu.VMEM((_GTAIL, D), jnp.float32),
        pltpu.VMEM((_GTAIL, D), jnp.float32),
        pltpu.VMEM((_GTAIL, D), jnp.float32),
        pltpu.SemaphoreType.DMA,
        pltpu.SemaphoreType.DMA,
    ],
)(_gather_body)


_ETILE = 640


def _edge_body(ea_ref, ks_ref, qd_ref, vs_ref, dst_ref, we_ref, be_ref,
               msg3_out, scp_out):
    eh = jnp.dot(ea_ref[...], we_ref[...], preferred_element_type=jnp.float32) + be_ref[...]
    pe = ks_ref[...] * qd_ref[...] * eh
    d_i = lax.broadcasted_iota(jnp.int32, (D, H), 0)
    h_i = lax.broadcasted_iota(jnp.int32, (D, H), 1)
    seg = jnp.where(d_i // DH == h_i, SCALE, 0.0)
    raw = jnp.dot(pe, seg, preferred_element_type=jnp.float32)
    sc = jnp.exp(jnp.clip(raw, -5.0, 5.0))
    d_i2 = lax.broadcasted_iota(jnp.int32, (H, D), 1)
    h_i2 = lax.broadcasted_iota(jnp.int32, (H, D), 0)
    rep = jnp.where(d_i2 // DH == h_i2, 1.0, 0.0)
    msg = vs_ref[...] * jnp.dot(sc, rep, preferred_element_type=jnp.float32)
    msg3_out[0] = msg[:, :D // 2]
    msg3_out[1] = msg[:, D // 2:]
    dd = dst_ref[...]
    k_i = lax.broadcasted_iota(jnp.int32, (_ETILE, 8), 1)
    oh = jnp.where(dd - (dd // 8) * 8 == k_i, 1.0, 0.0)
    k_i2 = lax.broadcasted_iota(jnp.int32, (8, _HD), 0)
    d_i3 = lax.broadcasted_iota(jnp.int32, (8, _HD), 1)
    r8 = jnp.where(d_i3 // (2 * H) == k_i2, 1.0, 0.0)
    ohrep = jnp.dot(oh, r8, preferred_element_type=jnp.float32)
    sc_pad = jnp.concatenate([sc, jnp.zeros_like(sc)], axis=1)
    sc_tile = jnp.concatenate([sc_pad] * 8, axis=1)
    scp_out[...] = sc_tile * ohrep


def _edge(edge_attr, ks, qd, vs, dst2d, we, be):
    grid = (E_EDGES // _ETILE,)
    rspec = pl.BlockSpec((_ETILE, D), lambda i: (i, 0))
    return pl.pallas_call(
        _edge_body,
        grid=grid,
        in_specs=[rspec, rspec, rspec, rspec,
                  pl.BlockSpec((_ETILE, 1), lambda i: (i, 0)),
                  pl.BlockSpec((D, D), lambda i: (0, 0)),
                  pl.BlockSpec((1, D), lambda i: (0, 0))],
        out_specs=[pl.BlockSpec((NC, _ETILE, D // 2), lambda i: (0, i, 0)),
                   pl.BlockSpec((_ETILE, _HD), lambda i: (i, 0))],
        out_shape=[jax.ShapeDtypeStruct((NC, E_EDGES, D // 2), jnp.float32),
                   jax.ShapeDtypeStruct((E_EDGES, _HD), jnp.float32)],
    )(edge_attr, ks, qd, vs, dst2d, we, be)


_EPS = E_EDGES // NS
_SCH = 128
_SFULL = _EPS // _SCH
_STAIL = _EPS - _SFULL * _SCH
_NPAD = 10240
_RPS = _NPAD // NS
_HD = D // 2
_ZW = 2 * H
_NZ = _NPAD // 8
_ZRPS = _NZ // NS


def _scatter_body(msg3, scp, dst, dst8, wv3_out, z_out,
                  acc, zacc, idx_v, idx2_v, dat_v, zdat_v,
                  idxt_v, idxt2_v, idxz_v, idxz2_v, sem):
    c = lax.axis_index("c")
    s = lax.axis_index("s")
    zero16 = jnp.zeros((16,), jnp.float32)

    def zbody(t, carry):
        r = t // 8
        j = t - r * 8
        dat_v[r, pl.ds(j * 16, 16)] = zero16
        return carry

    lax.fori_loop(0, _SCH * 8, zbody, 0)

    def set_block_indices(kk):
        base_r = (s * 5 + kk) * _SCH
        for j in range(8):
            idxz_v[pl.ds(j * 16, 16)] = base_r + j * 16 + lax.iota(jnp.int32, 16)

    def set_z_indices():
        base_r = s * _ZRPS
        for j in range(_ZRPS // 16):
            idxz2_v[pl.ds(j * 16, 16)] = base_r + j * 16 + lax.iota(jnp.int32, 16)

    for kk in range(5):
        set_block_indices(kk)
        pltpu.sync_copy(dat_v, acc.at[idxz_v])
    set_z_indices()
    pltpu.sync_copy(dat_v.at[pl.ds(0, _ZRPS)], zacc.at[idxz2_v])
    plsc.subcore_barrier()

    base = s * _EPS

    def do_chunk(off, idxb, idx2b, datb, zdatb, n):
        r1 = pltpu.async_copy(dst.at[pl.ds(off, n)], idxb, sem)
        r2 = pltpu.async_copy(dst8.at[pl.ds(off, n)], idx2b, sem)
        r3 = pltpu.async_copy(msg3.at[c, pl.ds(off, n)], datb, sem)
        r4 = pltpu.async_copy(scp.at[pl.ds(off, n)], zdatb, sem)
        r1.wait()
        r2.wait()
        r3.wait()
        r4.wait()
        a1 = pltpu.async_copy(datb, acc.at[idxb], sem, add=True)
        a2 = pltpu.async_copy(zdatb, zacc.at[idx2b], sem, add=True)
        a1.wait()
        a2.wait()

    def body(i, carry):
        do_chunk(base + i * _SCH, idx_v, idx2_v, dat_v, zdat_v, _SCH)
        return carry

    lax.fori_loop(0, _SFULL, body, 0)
    do_chunk(base + _SFULL * _SCH, idxt_v, idxt2_v,
             dat_v.at[pl.ds(0, _STAIL)], zdat_v.at[pl.ds(0, _STAIL)], _STAIL)
    plsc.subcore_barrier()

    for kk in range(5):
        set_block_indices(kk)
        row = (s * 5 + kk) * _SCH
        pltpu.async_copy(acc.at[idxz_v], dat_v, sem).wait()
        pltpu.sync_copy(dat_v, wv3_out.at[c, pl.ds(row, _SCH)])
    set_z_indices()
    pltpu.async_copy(zacc.at[idxz2_v], dat_v.at[pl.ds(0, _ZRPS)], sem).wait()
    pltpu.sync_copy(dat_v.at[pl.ds(0, _ZRPS)], z_out.at[pl.ds(s * _ZRPS, _ZRPS)])


_scatter = functools.partial(
    pl.kernel,
    mesh=plsc.VectorSubcoreMesh(core_axis_name="c", subcore_axis_name="s",
                                num_cores=NC, num_subcores=NS),
    out_type=[jax.ShapeDtypeStruct((NC, _NPAD, _HD), jnp.float32),
              jax.ShapeDtypeStruct((_NZ, _HD), jnp.float32)],
    scratch_types=[
        pltpu.VMEM_SHARED((_NPAD, _HD), jnp.float32),
        pltpu.VMEM_SHARED((_NZ, _HD), jnp.float32),
        pltpu.VMEM((_SCH,), jnp.int32),
        pltpu.VMEM((_SCH,), jnp.int32),
        pltpu.VMEM((_SCH, _HD), jnp.float32),
        pltpu.VMEM((_SCH, _HD), jnp.float32),
        pltpu.VMEM((_STAIL,), jnp.int32),
        pltpu.VMEM((_STAIL,), jnp.int32),
        pltpu.VMEM((_SCH,), jnp.int32),
        pltpu.VMEM((_ZRPS,), jnp.int32),
        pltpu.SemaphoreType.DMA,
    ],
)(_scatter_body)


_FTILE = 1000


def _fin_body(wva_ref, wvb_ref, z_ref, out_ref):
    h_i = lax.broadcasted_iota(jnp.int32, (_ZW, _HD), 0)
    d_i = lax.broadcasted_iota(jnp.int32, (_ZW, _HD), 1)
    rep_a = jnp.where(h_i == d_i // DH, 1.0, 0.0)
    rep_b = jnp.where(h_i == d_i // DH + H // 2, 1.0, 0.0)
    za = jnp.dot(z_ref[...], rep_a, preferred_element_type=jnp.float32)
    zb = jnp.dot(z_ref[...], rep_b, preferred_element_type=jnp.float32)
    out_ref[:, :_HD] = wva_ref[...] / (za + 1e-6)
    out_ref[:, _HD:] = wvb_ref[...] / (zb + 1e-6)


def _fin(wva, wvb, z):
    grid = (N_NODES // _FTILE,)
    return pl.pallas_call(
        _fin_body,
        grid=grid,
        in_specs=[pl.BlockSpec((_FTILE, _HD), lambda i: (i, 0)),
                  pl.BlockSpec((_FTILE, _HD), lambda i: (i, 0)),
                  pl.BlockSpec((_FTILE, _ZW), lambda i: (i, 0))],
        out_specs=pl.BlockSpec((_FTILE, D), lambda i: (i, 0)),
        out_shape=jax.ShapeDtypeStruct((N_NODES, D), jnp.float32),
    )(wva, wvb, z)


def kernel(x, edge_attr, Wq, bq, Wk, bk, We, be, Wv, bv, edge_index):
    src = edge_index[0]
    dst = edge_index[1]
    q, k, v = _proj(x, Wq, bq.reshape(1, D), Wk, bk.reshape(1, D),
                    Wv, bv.reshape(1, D))
    ks, qd, vs = _gather(k, q, v, src, dst)
    msg3, scp = _edge(edge_attr, ks, qd, vs, dst.reshape(E_EDGES, 1),
                      We, be.reshape(1, D))
    wv3, z2 = _scatter(msg3, scp, dst, dst // 8)
    return _fin(wv3[0], wv3[1], z2.reshape(_NPAD, _ZW))


# speedup vs baseline: 18.9288x; 1.0355x over previous
import functools

import jax
import jax.numpy as jnp
import numpy as np
from jax import lax
from jax.experimental import pallas as pl
from jax.experimental.pallas import tpu as pltpu
from jax.experimental.pallas import tpu_sc as plsc

N_NODES = 10000
E_EDGES = 160000
D = 256
H = 8
DH = 32
SCALE = 1.0 / np.sqrt(DH)

NC = 2
NS = 16
NW = NC * NS

_PROJ_TILE = 1000


def _proj_body(x_ref, wq, bq, wk, bk, wv, bv, q_out, k_out, v_out):
    x = x_ref[...]
    q_out[...] = jnp.dot(x, wq[...], preferred_element_type=jnp.float32) + bq[...]
    k_out[...] = jnp.dot(x, wk[...], preferred_element_type=jnp.float32) + bk[...]
    v_out[...] = jnp.dot(x, wv[...], preferred_element_type=jnp.float32) + bv[...]


def _proj(x, wq, bq, wk, bk, wv, bv):
    grid = (N_NODES // _PROJ_TILE,)
    wspec = pl.BlockSpec((D, D), lambda i: (0, 0))
    bspec = pl.BlockSpec((1, D), lambda i: (0, 0))
    rspec = pl.BlockSpec((_PROJ_TILE, D), lambda i: (i, 0))
    return pl.pallas_call(
        _proj_body,
        grid=grid,
        in_specs=[rspec, wspec, bspec, wspec, bspec, wspec, bspec],
        out_specs=[rspec, rspec, rspec],
        out_shape=[jax.ShapeDtypeStruct((N_NODES, D), jnp.float32)] * 3,
    )(x, wq, bq, wk, bk, wv, bv)


_EPW = E_EDGES // NW
_GCH = 64
_GNCH = _EPW // _GCH
_GPAIR = (_GNCH - 2) // 2
_GTAIL = _EPW - _GNCH * _GCH


def _gather_body(ktab, qtab, vtab, src, dst, ks_out, qd_out, vs_out,
                 i0s, i0d, b0k, b0q, b0v, i1s, i1d, b1k, b1q, b1v,
                 its, itd, btk, btq, btv,
                 g0, g1, w0, w1, gt):
    wid = lax.axis_index("s") * NC + lax.axis_index("c")
    base = wid * _EPW

    sets = ((i0s, i0d, b0k, b0q, b0v, g0, w0),
            (i1s, i1d, b1k, b1q, b1v, g1, w1))

    def rd(b, off):
        isv, idv, bk, bq, bv, g, _ = sets[b]
        pltpu.sync_copy(src.at[pl.ds(off, _GCH)], isv)
        pltpu.sync_copy(dst.at[pl.ds(off, _GCH)], idv)
        pltpu.async_copy(ktab.at[isv], bk, g)
        pltpu.async_copy(qtab.at[idv], bq, g)
        pltpu.async_copy(vtab.at[isv], bv, g)

    def dgwr(b, off):
        isv, idv, bk, bq, bv, g, w = sets[b]
        pltpu.make_async_copy(ktab.at[isv], bk, g).wait()
        pltpu.make_async_copy(qtab.at[idv], bq, g).wait()
        pltpu.make_async_copy(vtab.at[isv], bv, g).wait()
        pltpu.async_copy(bk, ks_out.at[pl.ds(off, _GCH)], w)
        pltpu.async_copy(bq, qd_out.at[pl.ds(off, _GCH)], w)
        pltpu.async_copy(bv, vs_out.at[pl.ds(off, _GCH)], w)

    def ww(b, off):
        isv, idv, bk, bq, bv, g, w = sets[b]
        pltpu.make_async_copy(bk, ks_out.at[pl.ds(off, _GCH)], w).wait()
        pltpu.make_async_copy(bq, qd_out.at[pl.ds(off, _GCH)], w).wait()
        pltpu.make_async_copy(bv, vs_out.at[pl.ds(off, _GCH)], w).wait()

    rd(0, base)

    def body(i, carry):
        c0 = base + (2 * i) * _GCH
        c1 = c0 + _GCH
        c2 = c1 + _GCH
        rd(1, c1)
        dgwr(0, c0)
        ww(0, c0)
        rd(0, c2)
        dgwr(1, c1)
        ww(1, c1)
        return carry

    lax.fori_loop(0, _GPAIR, body, 0)

    c76 = base + (_GNCH - 2) * _GCH
    c77 = c76 + _GCH
    dgwr(0, c76)
    rd(1, c77)
    dgwr(1, c77)
    ww(0, c76)
    ww(1, c77)

    offt = base + _GNCH * _GCH
    pltpu.sync_copy(src.at[pl.ds(offt, _GTAIL)], its)
    pltpu.sync_copy(dst.at[pl.ds(offt, _GTAIL)], itd)
    ck = pltpu.async_copy(ktab.at[its], btk, gt)
    cq = pltpu.async_copy(qtab.at[itd], btq, gt)
    cv = pltpu.async_copy(vtab.at[its], btv, gt)
    ck.wait()
    cq.wait()
    cv.wait()
    pltpu.sync_copy(btk, ks_out.at[pl.ds(offt, _GTAIL)])
    pltpu.sync_copy(btq, qd_out.at[pl.ds(offt, _GTAIL)])
    pltpu.sync_copy(btv, vs_out.at[pl.ds(offt, _GTAIL)])


_gather = functools.partial(
    pl.kernel,
    mesh=plsc.VectorSubcoreMesh(core_axis_name="c", subcore_axis_name="s",
                                num_cores=NC, num_subcores=NS),
    out_type=[jax.ShapeDtypeStruct((E_EDGES, D), jnp.float32)] * 3,
    scratch_types=(
        [pltpu.VMEM((_GCH,), jnp.int32)] * 2
        + [pltpu.VMEM((_GCH, D), jnp.float32)] * 3
        + [pltpu.VMEM((_GCH,), jnp.int32)] * 2
        + [pltpu.VMEM((_GCH, D), jnp.float32)] * 3
        + [pltpu.VMEM((_GTAIL,), jnp.int32)] * 2
        + [pltpu.VMEM((_GTAIL, D), jnp.float32)] * 3
        + [pltpu.SemaphoreType.DMA] * 5
    ),
)(_gather_body)


_ETILE = 640


def _edge_body(ea_ref, ks_ref, qd_ref, vs_ref, dst_ref, we_ref, be_ref,
               msg3_out, scp_out):
    eh = jnp.dot(ea_ref[...], we_ref[...], preferred_element_type=jnp.float32) + be_ref[...]
    pe = ks_ref[...] * qd_ref[...] * eh
    d_i = lax.broadcasted_iota(jnp.int32, (D, H), 0)
    h_i = lax.broadcasted_iota(jnp.int32, (D, H), 1)
    seg = jnp.where(d_i // DH == h_i, SCALE, 0.0)
    raw = jnp.dot(pe, seg, preferred_element_type=jnp.float32)
    sc = jnp.exp(jnp.clip(raw, -5.0, 5.0))
    d_i2 = lax.broadcasted_iota(jnp.int32, (H, D), 1)
    h_i2 = lax.broadcasted_iota(jnp.int32, (H, D), 0)
    rep = jnp.where(d_i2 // DH == h_i2, 1.0, 0.0)
    msg = vs_ref[...] * jnp.dot(sc, rep, preferred_element_type=jnp.float32)
    msg3_out[0] = msg[:, :D // 2]
    msg3_out[1] = msg[:, D // 2:]
    dd = dst_ref[...]
    k_i = lax.broadcasted_iota(jnp.int32, (_ETILE, 8), 1)
    oh = jnp.where(dd - (dd // 8) * 8 == k_i, 1.0, 0.0)
    k_i2 = lax.broadcasted_iota(jnp.int32, (8, _HD), 0)
    d_i3 = lax.broadcasted_iota(jnp.int32, (8, _HD), 1)
    r8 = jnp.where(d_i3 // (2 * H) == k_i2, 1.0, 0.0)
    ohrep = jnp.dot(oh, r8, preferred_element_type=jnp.float32)
    sc_pad = jnp.concatenate([sc, jnp.zeros_like(sc)], axis=1)
    sc_tile = jnp.concatenate([sc_pad] * 8, axis=1)
    scp_out[...] = sc_tile * ohrep


def _edge(edge_attr, ks, qd, vs, dst2d, we, be):
    grid = (E_EDGES // _ETILE,)
    rspec = pl.BlockSpec((_ETILE, D), lambda i: (i, 0))
    return pl.pallas_call(
        _edge_body,
        grid=grid,
        in_specs=[rspec, rspec, rspec, rspec,
                  pl.BlockSpec((_ETILE, 1), lambda i: (i, 0)),
                  pl.BlockSpec((D, D), lambda i: (0, 0)),
                  pl.BlockSpec((1, D), lambda i: (0, 0))],
        out_specs=[pl.BlockSpec((NC, _ETILE, D // 2), lambda i: (0, i, 0)),
                   pl.BlockSpec((_ETILE, _HD), lambda i: (i, 0))],
        out_shape=[jax.ShapeDtypeStruct((NC, E_EDGES, D // 2), jnp.float32),
                   jax.ShapeDtypeStruct((E_EDGES, _HD), jnp.float32)],
    )(edge_attr, ks, qd, vs, dst2d, we, be)


_EPS = E_EDGES // NS
_SCH = 128
_SFULL = _EPS // _SCH
_STAIL = _EPS - _SFULL * _SCH
_NPAD = 10240
_RPS = _NPAD // NS
_HD = D // 2
_ZW = 2 * H
_NZ = _NPAD // 8
_ZRPS = _NZ // NS


def _scatter_body(msg3, scp, dst, dst8, wv3_out, z_out,
                  acc, zacc, idx_v, idx2_v, dat_v, zdat_v,
                  idxt_v, idxt2_v, idxz_v, idxz2_v, sem):
    c = lax.axis_index("c")
    s = lax.axis_index("s")
    zero16 = jnp.zeros((16,), jnp.float32)

    def zbody(t, carry):
        r = t // 8
        j = t - r * 8
        dat_v[r, pl.ds(j * 16, 16)] = zero16
        return carry

    lax.fori_loop(0, _SCH * 8, zbody, 0)

    def set_block_indices(kk):
        base_r = (s * 5 + kk) * _SCH
        for j in range(8):
            idxz_v[pl.ds(j * 16, 16)] = base_r + j * 16 + lax.iota(jnp.int32, 16)

    def set_z_indices():
        base_r = s * _ZRPS
        for j in range(_ZRPS // 16):
            idxz2_v[pl.ds(j * 16, 16)] = base_r + j * 16 + lax.iota(jnp.int32, 16)

    for kk in range(5):
        set_block_indices(kk)
        pltpu.sync_copy(dat_v, acc.at[idxz_v])
    set_z_indices()
    pltpu.sync_copy(dat_v.at[pl.ds(0, _ZRPS)], zacc.at[idxz2_v])
    plsc.subcore_barrier()

    base = s * _EPS

    def do_chunk(off, idxb, idx2b, datb, zdatb, n):
        r1 = pltpu.async_copy(dst.at[pl.ds(off, n)], idxb, sem)
        r2 = pltpu.async_copy(dst8.at[pl.ds(off, n)], idx2b, sem)
        r3 = pltpu.async_copy(msg3.at[c, pl.ds(off, n)], datb, sem)
        r4 = pltpu.async_copy(scp.at[pl.ds(off, n)], zdatb, sem)
        r1.wait()
        r2.wait()
        r3.wait()
        r4.wait()
        a1 = pltpu.async_copy(datb, acc.at[idxb], sem, add=True)
        a2 = pltpu.async_copy(zdatb, zacc.at[idx2b], sem, add=True)
        a1.wait()
        a2.wait()

    def body(i, carry):
        do_chunk(base + i * _SCH, idx_v, idx2_v, dat_v, zdat_v, _SCH)
        return carry

    lax.fori_loop(0, _SFULL, body, 0)
    do_chunk(base + _SFULL * _SCH, idxt_v, idxt2_v,
             dat_v.at[pl.ds(0, _STAIL)], zdat_v.at[pl.ds(0, _STAIL)], _STAIL)
    plsc.subcore_barrier()

    for kk in range(5):
        set_block_indices(kk)
        row = (s * 5 + kk) * _SCH
        pltpu.async_copy(acc.at[idxz_v], dat_v, sem).wait()
        pltpu.sync_copy(dat_v, wv3_out.at[c, pl.ds(row, _SCH)])
    set_z_indices()
    pltpu.async_copy(zacc.at[idxz2_v], dat_v.at[pl.ds(0, _ZRPS)], sem).wait()
    pltpu.sync_copy(dat_v.at[pl.ds(0, _ZRPS)], z_out.at[pl.ds(s * _ZRPS, _ZRPS)])


_scatter = functools.partial(
    pl.kernel,
    mesh=plsc.VectorSubcoreMesh(core_axis_name="c", subcore_axis_name="s",
                                num_cores=NC, num_subcores=NS),
    out_type=[jax.ShapeDtypeStruct((NC, _NPAD, _HD), jnp.float32),
              jax.ShapeDtypeStruct((_NZ, _HD), jnp.float32)],
    scratch_types=[
        pltpu.VMEM_SHARED((_NPAD, _HD), jnp.float32),
        pltpu.VMEM_SHARED((_NZ, _HD), jnp.float32),
        pltpu.VMEM((_SCH,), jnp.int32),
        pltpu.VMEM((_SCH,), jnp.int32),
        pltpu.VMEM((_SCH, _HD), jnp.float32),
        pltpu.VMEM((_SCH, _HD), jnp.float32),
        pltpu.VMEM((_STAIL,), jnp.int32),
        pltpu.VMEM((_STAIL,), jnp.int32),
        pltpu.VMEM((_SCH,), jnp.int32),
        pltpu.VMEM((_ZRPS,), jnp.int32),
        pltpu.SemaphoreType.DMA,
    ],
)(_scatter_body)


_FTILE = 1000


def _fin_body(wva_ref, wvb_ref, z_ref, out_ref):
    h_i = lax.broadcasted_iota(jnp.int32, (_ZW, _HD), 0)
    d_i = lax.broadcasted_iota(jnp.int32, (_ZW, _HD), 1)
    rep_a = jnp.where(h_i == d_i // DH, 1.0, 0.0)
    rep_b = jnp.where(h_i == d_i // DH + H // 2, 1.0, 0.0)
    za = jnp.dot(z_ref[...], rep_a, preferred_element_type=jnp.float32)
    zb = jnp.dot(z_ref[...], rep_b, preferred_element_type=jnp.float32)
    out_ref[:, :_HD] = wva_ref[...] / (za + 1e-6)
    out_ref[:, _HD:] = wvb_ref[...] / (zb + 1e-6)


def _fin(wva, wvb, z):
    grid = (N_NODES // _FTILE,)
    return pl.pallas_call(
        _fin_body,
        grid=grid,
        in_specs=[pl.BlockSpec((_FTILE, _HD), lambda i: (i, 0)),
                  pl.BlockSpec((_FTILE, _HD), lambda i: (i, 0)),
                  pl.BlockSpec((_FTILE, _ZW), lambda i: (i, 0))],
        out_specs=pl.BlockSpec((_FTILE, D), lambda i: (i, 0)),
        out_shape=jax.ShapeDtypeStruct((N_NODES, D), jnp.float32),
    )(wva, wvb, z)


def kernel(x, edge_attr, Wq, bq, Wk, bk, We, be, Wv, bv, edge_index):
    src = edge_index[0]
    dst = edge_index[1]
    q, k, v = _proj(x, Wq, bq.reshape(1, D), Wk, bk.reshape(1, D),
                    Wv, bv.reshape(1, D))
    ks, qd, vs = _gather(k, q, v, src, dst)
    msg3, scp = _edge(edge_attr, ks, qd, vs, dst.reshape(E_EDGES, 1),
                      We, be.reshape(1, D))
    wv3, z2 = _scatter(msg3, scp, dst, dst // 8)
    return _fin(wv3[0], wv3[1], z2.reshape(_NPAD, _ZW))


# speedup vs baseline: 20.4047x; 1.0780x over previous
import functools

import jax
import jax.numpy as jnp
import numpy as np
from jax import lax
from jax.experimental import pallas as pl
from jax.experimental.pallas import tpu as pltpu
from jax.experimental.pallas import tpu_sc as plsc

N_NODES = 10000
E_EDGES = 160000
D = 256
H = 8
DH = 32
SCALE = 1.0 / np.sqrt(DH)

NC = 2
NS = 16
NW = NC * NS

_PROJ_TILE = 1000


def _proj_body(x_ref, wq, bq, wk, bk, wv, bv, q_out, k_out, v_out):
    x = x_ref[...]
    q_out[...] = jnp.dot(x, wq[...], preferred_element_type=jnp.float32) + bq[...]
    k_out[...] = jnp.dot(x, wk[...], preferred_element_type=jnp.float32) + bk[...]
    v_out[...] = jnp.dot(x, wv[...], preferred_element_type=jnp.float32) + bv[...]


def _proj(x, wq, bq, wk, bk, wv, bv):
    grid = (N_NODES // _PROJ_TILE,)
    wspec = pl.BlockSpec((D, D), lambda i: (0, 0))
    bspec = pl.BlockSpec((1, D), lambda i: (0, 0))
    rspec = pl.BlockSpec((_PROJ_TILE, D), lambda i: (i, 0))
    return pl.pallas_call(
        _proj_body,
        grid=grid,
        in_specs=[rspec, wspec, bspec, wspec, bspec, wspec, bspec],
        out_specs=[rspec, rspec, rspec],
        out_shape=[jax.ShapeDtypeStruct((N_NODES, D), jnp.float32)] * 3,
    )(x, wq, bq, wk, bk, wv, bv)


_EPW = E_EDGES // NW
_GCH = 64
_GNCH = _EPW // _GCH
_GPAIR = (_GNCH - 2) // 2
_GTAIL = _EPW - _GNCH * _GCH


def _gather_body(ktab, qtab, vtab, src, dst, ks_out, qd_out, vs_out,
                 i0s, i0d, b0k, b0q, b0v, i1s, i1d, b1k, b1q, b1v,
                 its, itd, btk, btq, btv,
                 g0, g1, w0, w1, gt):
    wid = lax.axis_index("s") * NC + lax.axis_index("c")
    base = wid * _EPW

    sets = ((i0s, i0d, b0k, b0q, b0v, g0, w0),
            (i1s, i1d, b1k, b1q, b1v, g1, w1))

    def rd(b, off):
        isv, idv, bk, bq, bv, g, _ = sets[b]
        pltpu.sync_copy(src.at[pl.ds(off, _GCH)], isv)
        pltpu.sync_copy(dst.at[pl.ds(off, _GCH)], idv)
        pltpu.async_copy(ktab.at[isv], bk, g)
        pltpu.async_copy(qtab.at[idv], bq, g)
        pltpu.async_copy(vtab.at[isv], bv, g)

    def dgwr(b, off):
        isv, idv, bk, bq, bv, g, w = sets[b]
        pltpu.make_async_copy(ktab.at[isv], bk, g).wait()
        pltpu.make_async_copy(qtab.at[idv], bq, g).wait()
        pltpu.make_async_copy(vtab.at[isv], bv, g).wait()
        pltpu.async_copy(bk, ks_out.at[pl.ds(off, _GCH)], w)
        pltpu.async_copy(bq, qd_out.at[pl.ds(off, _GCH)], w)
        pltpu.async_copy(bv, vs_out.at[pl.ds(off, _GCH)], w)

    def ww(b, off):
        isv, idv, bk, bq, bv, g, w = sets[b]
        pltpu.make_async_copy(bk, ks_out.at[pl.ds(off, _GCH)], w).wait()
        pltpu.make_async_copy(bq, qd_out.at[pl.ds(off, _GCH)], w).wait()
        pltpu.make_async_copy(bv, vs_out.at[pl.ds(off, _GCH)], w).wait()

    rd(0, base)

    def body(i, carry):
        c0 = base + (2 * i) * _GCH
        c1 = c0 + _GCH
        c2 = c1 + _GCH
        rd(1, c1)
        dgwr(0, c0)
        ww(0, c0)
        rd(0, c2)
        dgwr(1, c1)
        ww(1, c1)
        return carry

    lax.fori_loop(0, _GPAIR, body, 0)

    c76 = base + (_GNCH - 2) * _GCH
    c77 = c76 + _GCH
    dgwr(0, c76)
    rd(1, c77)
    dgwr(1, c77)
    ww(0, c76)
    ww(1, c77)

    offt = base + _GNCH * _GCH
    pltpu.sync_copy(src.at[pl.ds(offt, _GTAIL)], its)
    pltpu.sync_copy(dst.at[pl.ds(offt, _GTAIL)], itd)
    ck = pltpu.async_copy(ktab.at[its], btk, gt)
    cq = pltpu.async_copy(qtab.at[itd], btq, gt)
    cv = pltpu.async_copy(vtab.at[its], btv, gt)
    ck.wait()
    cq.wait()
    cv.wait()
    pltpu.sync_copy(btk, ks_out.at[pl.ds(offt, _GTAIL)])
    pltpu.sync_copy(btq, qd_out.at[pl.ds(offt, _GTAIL)])
    pltpu.sync_copy(btv, vs_out.at[pl.ds(offt, _GTAIL)])


_gather = functools.partial(
    pl.kernel,
    mesh=plsc.VectorSubcoreMesh(core_axis_name="c", subcore_axis_name="s",
                                num_cores=NC, num_subcores=NS),
    out_type=[jax.ShapeDtypeStruct((E_EDGES, D), jnp.float32)] * 3,
    scratch_types=(
        [pltpu.VMEM((_GCH,), jnp.int32)] * 2
        + [pltpu.VMEM((_GCH, D), jnp.float32)] * 3
        + [pltpu.VMEM((_GCH,), jnp.int32)] * 2
        + [pltpu.VMEM((_GCH, D), jnp.float32)] * 3
        + [pltpu.VMEM((_GTAIL,), jnp.int32)] * 2
        + [pltpu.VMEM((_GTAIL, D), jnp.float32)] * 3
        + [pltpu.SemaphoreType.DMA] * 5
    ),
)(_gather_body)


_ETILE = 640


def _edge_body(ea_ref, ks_ref, qd_ref, vs_ref, dst_ref, we_ref, be_ref,
               msg3_out, scp_out):
    eh = jnp.dot(ea_ref[...], we_ref[...], preferred_element_type=jnp.float32) + be_ref[...]
    pe = ks_ref[...] * qd_ref[...] * eh
    d_i = lax.broadcasted_iota(jnp.int32, (D, H), 0)
    h_i = lax.broadcasted_iota(jnp.int32, (D, H), 1)
    seg = jnp.where(d_i // DH == h_i, SCALE, 0.0)
    raw = jnp.dot(pe, seg, preferred_element_type=jnp.float32)
    sc = jnp.exp(jnp.clip(raw, -5.0, 5.0))
    d_i2 = lax.broadcasted_iota(jnp.int32, (H, D), 1)
    h_i2 = lax.broadcasted_iota(jnp.int32, (H, D), 0)
    rep = jnp.where(d_i2 // DH == h_i2, 1.0, 0.0)
    msg = vs_ref[...] * jnp.dot(sc, rep, preferred_element_type=jnp.float32)
    msg3_out[0] = msg[:, :D // 2]
    msg3_out[1] = msg[:, D // 2:]
    dd = dst_ref[...]
    k_i = lax.broadcasted_iota(jnp.int32, (_ETILE, 8), 1)
    oh = jnp.where(dd - (dd // 8) * 8 == k_i, 1.0, 0.0)
    k_i2 = lax.broadcasted_iota(jnp.int32, (8, _HD), 0)
    d_i3 = lax.broadcasted_iota(jnp.int32, (8, _HD), 1)
    r8 = jnp.where(d_i3 // (2 * H) == k_i2, 1.0, 0.0)
    ohrep = jnp.dot(oh, r8, preferred_element_type=jnp.float32)
    sc_pad = jnp.concatenate([sc, jnp.zeros_like(sc)], axis=1)
    sc_tile = jnp.concatenate([sc_pad] * 8, axis=1)
    scp_out[...] = sc_tile * ohrep


def _edge(edge_attr, ks, qd, vs, dst2d, we, be):
    grid = (E_EDGES // _ETILE,)
    rspec = pl.BlockSpec((_ETILE, D), lambda i: (i, 0))
    return pl.pallas_call(
        _edge_body,
        grid=grid,
        in_specs=[rspec, rspec, rspec, rspec,
                  pl.BlockSpec((_ETILE, 1), lambda i: (i, 0)),
                  pl.BlockSpec((D, D), lambda i: (0, 0)),
                  pl.BlockSpec((1, D), lambda i: (0, 0))],
        out_specs=[pl.BlockSpec((NC, _ETILE, D // 2), lambda i: (0, i, 0)),
                   pl.BlockSpec((_ETILE, _HD), lambda i: (i, 0))],
        out_shape=[jax.ShapeDtypeStruct((NC, E_EDGES, D // 2), jnp.float32),
                   jax.ShapeDtypeStruct((E_EDGES, _HD), jnp.float32)],
    )(edge_attr, ks, qd, vs, dst2d, we, be)


_EPS = E_EDGES // NS
_SCH = 64
_SNCH = _EPS // _SCH
_SPAIR = (_SNCH - 2) // 2
_STAIL = _EPS - _SNCH * _SCH
_NPAD = 10240
_RPS = _NPAD // NS
_HD = D // 2
_ZW = 2 * H
_NZ = _NPAD // 8
_ZRPS = _NZ // NS


def _scatter_body(msg3, scp, dst, dst8, wv3_out, z_out,
                  acc, zacc, i0, i0b, d0, z0, i1, i1b, d1, z1,
                  it_, it2_, idxz_v, idxz16_v, r0, r1, a0, a1, st):
    c = lax.axis_index("c")
    s = lax.axis_index("s")
    zero16 = jnp.zeros((16,), jnp.float32)

    def zbody(t, carry):
        r = t // 8
        j = t - r * 8
        d0[r, pl.ds(j * 16, 16)] = zero16
        return carry

    lax.fori_loop(0, _SCH * 8, zbody, 0)

    def set_idxz(base_r):
        for j in range(4):
            idxz_v[pl.ds(j * 16, 16)] = base_r + j * 16 + lax.iota(jnp.int32, 16)

    for kk in range(10):
        set_idxz(s * _RPS + kk * _SCH)
        pltpu.sync_copy(d0, acc.at[idxz_v])
    set_idxz(s * _ZRPS)
    pltpu.sync_copy(d0, zacc.at[idxz_v])
    idxz16_v[...] = s * _ZRPS + _SCH + lax.iota(jnp.int32, 16)
    pltpu.sync_copy(d0.at[pl.ds(0, 16)], zacc.at[idxz16_v])
    plsc.subcore_barrier()

    base = s * _EPS
    sets = ((i0, i0b, d0, z0, r0, a0), (i1, i1b, d1, z1, r1, a1))

    def rd(b, off):
        ix, ix2, dd, zz, rs, _ = sets[b]
        pltpu.async_copy(dst.at[pl.ds(off, _SCH)], ix, rs)
        pltpu.async_copy(dst8.at[pl.ds(off, _SCH)], ix2, rs)
        pltpu.async_copy(msg3.at[c, pl.ds(off, _SCH)], dd, rs)
        pltpu.async_copy(scp.at[pl.ds(off, _SCH)], zz, rs)

    def add(b, off):
        ix, ix2, dd, zz, rs, asm = sets[b]
        pltpu.make_async_copy(dst.at[pl.ds(off, _SCH)], ix, rs).wait()
        pltpu.make_async_copy(dst8.at[pl.ds(off, _SCH)], ix2, rs).wait()
        pltpu.make_async_copy(msg3.at[c, pl.ds(off, _SCH)], dd, rs).wait()
        pltpu.make_async_copy(scp.at[pl.ds(off, _SCH)], zz, rs).wait()
        pltpu.async_copy(dd, acc.at[ix], asm, add=True)
        pltpu.async_copy(zz, zacc.at[ix2], asm, add=True)

    def wa(b):
        ix, ix2, dd, zz, rs, asm = sets[b]
        pltpu.make_async_copy(dd, acc.at[ix], asm).wait()
        pltpu.make_async_copy(zz, zacc.at[ix2], asm).wait()

    rd(0, base)

    def body(i, carry):
        c0 = base + (2 * i) * _SCH
        c1 = c0 + _SCH
        c2 = c1 + _SCH
        rd(1, c1)
        add(0, c0)
        wa(0)
        rd(0, c2)
        add(1, c1)
        wa(1)
        return carry

    lax.fori_loop(0, _SPAIR, body, 0)

    cA = base + (_SNCH - 2) * _SCH
    cB = cA + _SCH
    add(0, cA)
    rd(1, cB)
    add(1, cB)
    wa(0)
    wa(1)

    offt = base + _SNCH * _SCH
    t1 = pltpu.async_copy(dst.at[pl.ds(offt, _STAIL)], it_, st)
    t2 = pltpu.async_copy(dst8.at[pl.ds(offt, _STAIL)], it2_, st)
    t3 = pltpu.async_copy(msg3.at[c, pl.ds(offt, _STAIL)], d0.at[pl.ds(0, _STAIL)], st)
    t4 = pltpu.async_copy(scp.at[pl.ds(offt, _STAIL)], z0.at[pl.ds(0, _STAIL)], st)
    t1.wait()
    t2.wait()
    t3.wait()
    t4.wait()
    ta = pltpu.async_copy(d0.at[pl.ds(0, _STAIL)], acc.at[it_], st, add=True)
    tb = pltpu.async_copy(z0.at[pl.ds(0, _STAIL)], zacc.at[it2_], st, add=True)
    ta.wait()
    tb.wait()
    plsc.subcore_barrier()

    for kk in range(10):
        row = s * _RPS + kk * _SCH
        set_idxz(row)
        pltpu.async_copy(acc.at[idxz_v], d0, st).wait()
        pltpu.sync_copy(d0, wv3_out.at[c, pl.ds(row, _SCH)])
    set_idxz(s * _ZRPS)
    pltpu.async_copy(zacc.at[idxz_v], d0, st).wait()
    pltpu.sync_copy(d0, z_out.at[pl.ds(s * _ZRPS, _SCH)])
    pltpu.async_copy(zacc.at[idxz16_v], d0.at[pl.ds(0, 16)], st).wait()
    pltpu.sync_copy(d0.at[pl.ds(0, 16)], z_out.at[pl.ds(s * _ZRPS + _SCH, 16)])


_scatter = functools.partial(
    pl.kernel,
    mesh=plsc.VectorSubcoreMesh(core_axis_name="c", subcore_axis_name="s",
                                num_cores=NC, num_subcores=NS),
    out_type=[jax.ShapeDtypeStruct((NC, _NPAD, _HD), jnp.float32),
              jax.ShapeDtypeStruct((_NZ, _HD), jnp.float32)],
    scratch_types=(
        [pltpu.VMEM_SHARED((_NPAD, _HD), jnp.float32),
         pltpu.VMEM_SHARED((_NZ, _HD), jnp.float32)]
        + [pltpu.VMEM((_SCH,), jnp.int32), pltpu.VMEM((_SCH,), jnp.int32),
           pltpu.VMEM((_SCH, _HD), jnp.float32), pltpu.VMEM((_SCH, _HD), jnp.float32)] * 2
        + [pltpu.VMEM((_STAIL,), jnp.int32), pltpu.VMEM((_STAIL,), jnp.int32),
           pltpu.VMEM((_SCH,), jnp.int32), pltpu.VMEM((16,), jnp.int32)]
        + [pltpu.SemaphoreType.DMA] * 5
    ),
)(_scatter_body)


_FTILE = 1000


def _fin_body(wva_ref, wvb_ref, z_ref, out_ref):
    h_i = lax.broadcasted_iota(jnp.int32, (_ZW, _HD), 0)
    d_i = lax.broadcasted_iota(jnp.int32, (_ZW, _HD), 1)
    rep_a = jnp.where(h_i == d_i // DH, 1.0, 0.0)
    rep_b = jnp.where(h_i == d_i // DH + H // 2, 1.0, 0.0)
    za = jnp.dot(z_ref[...], rep_a, preferred_element_type=jnp.float32)
    zb = jnp.dot(z_ref[...], rep_b, preferred_element_type=jnp.float32)
    out_ref[:, :_HD] = wva_ref[...] / (za + 1e-6)
    out_ref[:, _HD:] = wvb_ref[...] / (zb + 1e-6)


def _fin(wva, wvb, z):
    grid = (N_NODES // _FTILE,)
    return pl.pallas_call(
        _fin_body,
        grid=grid,
        in_specs=[pl.BlockSpec((_FTILE, _HD), lambda i: (i, 0)),
                  pl.BlockSpec((_FTILE, _HD), lambda i: (i, 0)),
                  pl.BlockSpec((_FTILE, _ZW), lambda i: (i, 0))],
        out_specs=pl.BlockSpec((_FTILE, D), lambda i: (i, 0)),
        out_shape=jax.ShapeDtypeStruct((N_NODES, D), jnp.float32),
    )(wva, wvb, z)


def kernel(x, edge_attr, Wq, bq, Wk, bk, We, be, Wv, bv, edge_index):
    src = edge_index[0]
    dst = edge_index[1]
    q, k, v = _proj(x, Wq, bq.reshape(1, D), Wk, bk.reshape(1, D),
                    Wv, bv.reshape(1, D))
    ks, qd, vs = _gather(k, q, v, src, dst)
    msg3, scp = _edge(edge_attr, ks, qd, vs, dst.reshape(E_EDGES, 1),
                      We, be.reshape(1, D))
    wv3, z2 = _scatter(msg3, scp, dst, dst // 8)
    return _fin(wv3[0], wv3[1], z2.reshape(_NPAD, _ZW))


# speedup vs baseline: 22.2040x; 1.0882x over previous
import functools

import jax
import jax.numpy as jnp
import numpy as np
from jax import lax
from jax.experimental import pallas as pl
from jax.experimental.pallas import tpu as pltpu
from jax.experimental.pallas import tpu_sc as plsc

N_NODES = 10000
E_EDGES = 160000
D = 256
H = 8
DH = 32
SCALE = 1.0 / np.sqrt(DH)

NC = 2
NS = 16
NW = NC * NS

_PROJ_TILE = 2000


def _proj_body(x_ref, wq, bq, wk, bk, wv, bv, q_out, k_out, v_out):
    x = x_ref[...]
    q_out[...] = jnp.dot(x, wq[...], preferred_element_type=jnp.float32) + bq[...]
    k_out[...] = jnp.dot(x, wk[...], preferred_element_type=jnp.float32) + bk[...]
    v_out[...] = jnp.dot(x, wv[...], preferred_element_type=jnp.float32) + bv[...]


def _proj(x, wq, bq, wk, bk, wv, bv):
    grid = (N_NODES // _PROJ_TILE,)
    wspec = pl.BlockSpec((D, D), lambda i: (0, 0))
    bspec = pl.BlockSpec((1, D), lambda i: (0, 0))
    rspec = pl.BlockSpec((_PROJ_TILE, D), lambda i: (i, 0))
    return pl.pallas_call(
        _proj_body,
        grid=grid,
        in_specs=[rspec, wspec, bspec, wspec, bspec, wspec, bspec],
        out_specs=[rspec, rspec, rspec],
        out_shape=[jax.ShapeDtypeStruct((N_NODES, D), jnp.float32)] * 3,
    )(x, wq, bq, wk, bk, wv, bv)


_EPW = E_EDGES // NW
_GCH = 64
_GNCH = _EPW // _GCH
_GPAIR = (_GNCH - 2) // 2
_GTAIL = _EPW - _GNCH * _GCH


def _gather_body(ktab, qtab, vtab, src, dst, ks_out, qd_out, vs_out,
                 i0s, i0d, b0k, b0q, b0v, i1s, i1d, b1k, b1q, b1v,
                 its, itd, btk, btq, btv,
                 g0, g1, w0, w1, gt):
    wid = lax.axis_index("s") * NC + lax.axis_index("c")
    base = wid * _EPW

    sets = ((i0s, i0d, b0k, b0q, b0v, g0, w0),
            (i1s, i1d, b1k, b1q, b1v, g1, w1))

    def rd(b, off):
        isv, idv, bk, bq, bv, g, _ = sets[b]
        pltpu.sync_copy(src.at[pl.ds(off, _GCH)], isv)
        pltpu.sync_copy(dst.at[pl.ds(off, _GCH)], idv)
        pltpu.async_copy(ktab.at[isv], bk, g)
        pltpu.async_copy(qtab.at[idv], bq, g)
        pltpu.async_copy(vtab.at[isv], bv, g)

    def dgwr(b, off):
        isv, idv, bk, bq, bv, g, w = sets[b]
        pltpu.make_async_copy(ktab.at[isv], bk, g).wait()
        pltpu.make_async_copy(qtab.at[idv], bq, g).wait()
        pltpu.make_async_copy(vtab.at[isv], bv, g).wait()
        pltpu.async_copy(bk, ks_out.at[pl.ds(off, _GCH)], w)
        pltpu.async_copy(bq, qd_out.at[pl.ds(off, _GCH)], w)
        pltpu.async_copy(bv, vs_out.at[pl.ds(off, _GCH)], w)

    def ww(b, off):
        isv, idv, bk, bq, bv, g, w = sets[b]
        pltpu.make_async_copy(bk, ks_out.at[pl.ds(off, _GCH)], w).wait()
        pltpu.make_async_copy(bq, qd_out.at[pl.ds(off, _GCH)], w).wait()
        pltpu.make_async_copy(bv, vs_out.at[pl.ds(off, _GCH)], w).wait()

    rd(0, base)

    def body(i, carry):
        c0 = base + (2 * i) * _GCH
        c1 = c0 + _GCH
        c2 = c1 + _GCH
        rd(1, c1)
        dgwr(0, c0)
        ww(0, c0)
        rd(0, c2)
        dgwr(1, c1)
        ww(1, c1)
        return carry

    lax.fori_loop(0, _GPAIR, body, 0)

    c76 = base + (_GNCH - 2) * _GCH
    c77 = c76 + _GCH
    dgwr(0, c76)
    rd(1, c77)
    dgwr(1, c77)
    ww(0, c76)
    ww(1, c77)

    offt = base + _GNCH * _GCH
    pltpu.sync_copy(src.at[pl.ds(offt, _GTAIL)], its)
    pltpu.sync_copy(dst.at[pl.ds(offt, _GTAIL)], itd)
    ck = pltpu.async_copy(ktab.at[its], btk, gt)
    cq = pltpu.async_copy(qtab.at[itd], btq, gt)
    cv = pltpu.async_copy(vtab.at[its], btv, gt)
    ck.wait()
    cq.wait()
    cv.wait()
    pltpu.sync_copy(btk, ks_out.at[pl.ds(offt, _GTAIL)])
    pltpu.sync_copy(btq, qd_out.at[pl.ds(offt, _GTAIL)])
    pltpu.sync_copy(btv, vs_out.at[pl.ds(offt, _GTAIL)])


_gather = functools.partial(
    pl.kernel,
    mesh=plsc.VectorSubcoreMesh(core_axis_name="c", subcore_axis_name="s",
                                num_cores=NC, num_subcores=NS),
    out_type=[jax.ShapeDtypeStruct((E_EDGES, D), jnp.float32)] * 3,
    scratch_types=(
        [pltpu.VMEM((_GCH,), jnp.int32)] * 2
        + [pltpu.VMEM((_GCH, D), jnp.float32)] * 3
        + [pltpu.VMEM((_GCH,), jnp.int32)] * 2
        + [pltpu.VMEM((_GCH, D), jnp.float32)] * 3
        + [pltpu.VMEM((_GTAIL,), jnp.int32)] * 2
        + [pltpu.VMEM((_GTAIL, D), jnp.float32)] * 3
        + [pltpu.SemaphoreType.DMA] * 5
    ),
)(_gather_body)


_ETILE = 1280


def _edge_body(ea_ref, ks_ref, qd_ref, vs_ref, dst_ref, we_ref, be_ref,
               msg3_out, scp_out):
    eh = jnp.dot(ea_ref[...], we_ref[...], preferred_element_type=jnp.float32) + be_ref[...]
    pe = ks_ref[...] * qd_ref[...] * eh
    d_i = lax.broadcasted_iota(jnp.int32, (D, H), 0)
    h_i = lax.broadcasted_iota(jnp.int32, (D, H), 1)
    seg = jnp.where(d_i // DH == h_i, SCALE, 0.0)
    raw = jnp.dot(pe, seg, preferred_element_type=jnp.float32)
    sc = jnp.exp(jnp.clip(raw, -5.0, 5.0))
    d_i2 = lax.broadcasted_iota(jnp.int32, (H, D), 1)
    h_i2 = lax.broadcasted_iota(jnp.int32, (H, D), 0)
    rep = jnp.where(d_i2 // DH == h_i2, 1.0, 0.0)
    msg = vs_ref[...] * jnp.dot(sc, rep, preferred_element_type=jnp.float32)
    msg3_out[0] = msg[:, :D // 2]
    msg3_out[1] = msg[:, D // 2:]
    dd = dst_ref[...]
    k_i = lax.broadcasted_iota(jnp.int32, (_ETILE, 8), 1)
    oh = jnp.where(dd - (dd // 8) * 8 == k_i, 1.0, 0.0)
    k_i2 = lax.broadcasted_iota(jnp.int32, (8, _HD), 0)
    d_i3 = lax.broadcasted_iota(jnp.int32, (8, _HD), 1)
    r8 = jnp.where(d_i3 // (2 * H) == k_i2, 1.0, 0.0)
    ohrep = jnp.dot(oh, r8, preferred_element_type=jnp.float32)
    sc_pad = jnp.concatenate([sc, jnp.zeros_like(sc)], axis=1)
    sc_tile = jnp.concatenate([sc_pad] * 8, axis=1)
    scp_out[...] = sc_tile * ohrep


def _edge(edge_attr, ks, qd, vs, dst2d, we, be):
    grid = (E_EDGES // _ETILE,)
    rspec = pl.BlockSpec((_ETILE, D), lambda i: (i, 0))
    return pl.pallas_call(
        _edge_body,
        grid=grid,
        in_specs=[rspec, rspec, rspec, rspec,
                  pl.BlockSpec((_ETILE, 1), lambda i: (i, 0)),
                  pl.BlockSpec((D, D), lambda i: (0, 0)),
                  pl.BlockSpec((1, D), lambda i: (0, 0))],
        out_specs=[pl.BlockSpec((NC, _ETILE, D // 2), lambda i: (0, i, 0)),
                   pl.BlockSpec((_ETILE, _HD), lambda i: (i, 0))],
        out_shape=[jax.ShapeDtypeStruct((NC, E_EDGES, D // 2), jnp.float32),
                   jax.ShapeDtypeStruct((E_EDGES, _HD), jnp.float32)],
    )(edge_attr, ks, qd, vs, dst2d, we, be)


_EPS = E_EDGES // NS
_SCH = 64
_SNCH = _EPS // _SCH
_SPAIR = (_SNCH - 2) // 2
_STAIL = _EPS - _SNCH * _SCH
_NPAD = 10240
_RPS = _NPAD // NS
_HD = D // 2
_ZW = 2 * H
_NZ = _NPAD // 8
_ZRPS = _NZ // NS


def _scatter_body(msg3, scp, dst, dst8, wv3_out, z_out,
                  acc, zacc, i0, i0b, d0, z0, i1, i1b, d1, z1,
                  it_, it2_, idxz_v, idxz16_v, r0, r1, a0, a1, st):
    c = lax.axis_index("c")
    s = lax.axis_index("s")
    zero16 = jnp.zeros((16,), jnp.float32)

    def zbody(t, carry):
        r = t // 8
        j = t - r * 8
        d0[r, pl.ds(j * 16, 16)] = zero16
        return carry

    lax.fori_loop(0, _SCH * 8, zbody, 0)

    def set_idxz(base_r):
        for j in range(4):
            idxz_v[pl.ds(j * 16, 16)] = base_r + j * 16 + lax.iota(jnp.int32, 16)

    for kk in range(10):
        set_idxz(s * _RPS + kk * _SCH)
        pltpu.sync_copy(d0, acc.at[idxz_v])
    set_idxz(s * _ZRPS)
    pltpu.sync_copy(d0, zacc.at[idxz_v])
    idxz16_v[...] = s * _ZRPS + _SCH + lax.iota(jnp.int32, 16)
    pltpu.sync_copy(d0.at[pl.ds(0, 16)], zacc.at[idxz16_v])
    plsc.subcore_barrier()

    base = s * _EPS
    sets = ((i0, i0b, d0, z0, r0, a0), (i1, i1b, d1, z1, r1, a1))

    def rd(b, off):
        ix, ix2, dd, zz, rs, _ = sets[b]
        pltpu.async_copy(dst.at[pl.ds(off, _SCH)], ix, rs)
        pltpu.async_copy(dst8.at[pl.ds(off, _SCH)], ix2, rs)
        pltpu.async_copy(msg3.at[c, pl.ds(off, _SCH)], dd, rs)
        pltpu.async_copy(scp.at[pl.ds(off, _SCH)], zz, rs)

    def add(b, off):
        ix, ix2, dd, zz, rs, asm = sets[b]
        pltpu.make_async_copy(dst.at[pl.ds(off, _SCH)], ix, rs).wait()
        pltpu.make_async_copy(dst8.at[pl.ds(off, _SCH)], ix2, rs).wait()
        pltpu.make_async_copy(msg3.at[c, pl.ds(off, _SCH)], dd, rs).wait()
        pltpu.make_async_copy(scp.at[pl.ds(off, _SCH)], zz, rs).wait()
        pltpu.async_copy(dd, acc.at[ix], asm, add=True)
        pltpu.async_copy(zz, zacc.at[ix2], asm, add=True)

    def wa(b):
        ix, ix2, dd, zz, rs, asm = sets[b]
        pltpu.make_async_copy(dd, acc.at[ix], asm).wait()
        pltpu.make_async_copy(zz, zacc.at[ix2], asm).wait()

    rd(0, base)

    def body(i, carry):
        c0 = base + (2 * i) * _SCH
        c1 = c0 + _SCH
        c2 = c1 + _SCH
        rd(1, c1)
        add(0, c0)
        wa(0)
        rd(0, c2)
        add(1, c1)
        wa(1)
        return carry

    lax.fori_loop(0, _SPAIR, body, 0)

    cA = base + (_SNCH - 2) * _SCH
    cB = cA + _SCH
    add(0, cA)
    rd(1, cB)
    add(1, cB)
    wa(0)
    wa(1)

    offt = base + _SNCH * _SCH
    t1 = pltpu.async_copy(dst.at[pl.ds(offt, _STAIL)], it_, st)
    t2 = pltpu.async_copy(dst8.at[pl.ds(offt, _STAIL)], it2_, st)
    t3 = pltpu.async_copy(msg3.at[c, pl.ds(offt, _STAIL)], d0.at[pl.ds(0, _STAIL)], st)
    t4 = pltpu.async_copy(scp.at[pl.ds(offt, _STAIL)], z0.at[pl.ds(0, _STAIL)], st)
    t1.wait()
    t2.wait()
    t3.wait()
    t4.wait()
    ta = pltpu.async_copy(d0.at[pl.ds(0, _STAIL)], acc.at[it_], st, add=True)
    tb = pltpu.async_copy(z0.at[pl.ds(0, _STAIL)], zacc.at[it2_], st, add=True)
    ta.wait()
    tb.wait()
    plsc.subcore_barrier()

    for kk in range(10):
        row = s * _RPS + kk * _SCH
        set_idxz(row)
        pltpu.async_copy(acc.at[idxz_v], d0, st).wait()
        pltpu.sync_copy(d0, wv3_out.at[c, pl.ds(row, _SCH)])
    set_idxz(s * _ZRPS)
    pltpu.async_copy(zacc.at[idxz_v], d0, st).wait()
    pltpu.sync_copy(d0, z_out.at[pl.ds(s * _ZRPS, _SCH)])
    pltpu.async_copy(zacc.at[idxz16_v], d0.at[pl.ds(0, 16)], st).wait()
    pltpu.sync_copy(d0.at[pl.ds(0, 16)], z_out.at[pl.ds(s * _ZRPS + _SCH, 16)])


_scatter = functools.partial(
    pl.kernel,
    mesh=plsc.VectorSubcoreMesh(core_axis_name="c", subcore_axis_name="s",
                                num_cores=NC, num_subcores=NS),
    out_type=[jax.ShapeDtypeStruct((NC, _NPAD, _HD), jnp.float32),
              jax.ShapeDtypeStruct((_NZ, _HD), jnp.float32)],
    scratch_types=(
        [pltpu.VMEM_SHARED((_NPAD, _HD), jnp.float32),
         pltpu.VMEM_SHARED((_NZ, _HD), jnp.float32)]
        + [pltpu.VMEM((_SCH,), jnp.int32), pltpu.VMEM((_SCH,), jnp.int32),
           pltpu.VMEM((_SCH, _HD), jnp.float32), pltpu.VMEM((_SCH, _HD), jnp.float32)] * 2
        + [pltpu.VMEM((_STAIL,), jnp.int32), pltpu.VMEM((_STAIL,), jnp.int32),
           pltpu.VMEM((_SCH,), jnp.int32), pltpu.VMEM((16,), jnp.int32)]
        + [pltpu.SemaphoreType.DMA] * 5
    ),
)(_scatter_body)


_FTILE = 1000


def _fin_body(wva_ref, wvb_ref, z_ref, out_ref):
    h_i = lax.broadcasted_iota(jnp.int32, (_ZW, _HD), 0)
    d_i = lax.broadcasted_iota(jnp.int32, (_ZW, _HD), 1)
    rep_a = jnp.where(h_i == d_i // DH, 1.0, 0.0)
    rep_b = jnp.where(h_i == d_i // DH + H // 2, 1.0, 0.0)
    za = jnp.dot(z_ref[...], rep_a, preferred_element_type=jnp.float32)
    zb = jnp.dot(z_ref[...], rep_b, preferred_element_type=jnp.float32)
    out_ref[:, :_HD] = wva_ref[...] / (za + 1e-6)
    out_ref[:, _HD:] = wvb_ref[...] / (zb + 1e-6)


def _fin(wva, wvb, z):
    grid = (N_NODES // _FTILE,)
    return pl.pallas_call(
        _fin_body,
        grid=grid,
        in_specs=[pl.BlockSpec((_FTILE, _HD), lambda i: (i, 0)),
                  pl.BlockSpec((_FTILE, _HD), lambda i: (i, 0)),
                  pl.BlockSpec((_FTILE, _ZW), lambda i: (i, 0))],
        out_specs=pl.BlockSpec((_FTILE, D), lambda i: (i, 0)),
        out_shape=jax.ShapeDtypeStruct((N_NODES, D), jnp.float32),
    )(wva, wvb, z)


def kernel(x, edge_attr, Wq, bq, Wk, bk, We, be, Wv, bv, edge_index):
    src = edge_index[0]
    dst = edge_index[1]
    q, k, v = _proj(x, Wq, bq.reshape(1, D), Wk, bk.reshape(1, D),
                    Wv, bv.reshape(1, D))
    ks, qd, vs = _gather(k, q, v, src, dst)
    msg3, scp = _edge(edge_attr, ks, qd, vs, dst.reshape(E_EDGES, 1),
                      We, be.reshape(1, D))
    wv3, z2 = _scatter(msg3, scp, dst, dst // 8)
    return _fin(wv3[0], wv3[1], z2.reshape(_NPAD, _ZW))


# speedup vs baseline: 22.7693x; 1.0255x over previous
import functools

import jax
import jax.numpy as jnp
import numpy as np
from jax import lax
from jax.experimental import pallas as pl
from jax.experimental.pallas import tpu as pltpu
from jax.experimental.pallas import tpu_sc as plsc

N_NODES = 10000
E_EDGES = 160000
D = 256
H = 8
DH = 32
SCALE = 1.0 / np.sqrt(DH)

NC = 2
NS = 16
NW = NC * NS

_PROJ_TILE = 2000


def _proj_body(x_ref, wq, bq, wk, bk, wv, bv, q_out, k_out, v_out):
    x = x_ref[...]
    q_out[...] = jnp.dot(x, wq[...], preferred_element_type=jnp.float32) + bq[...]
    k_out[...] = jnp.dot(x, wk[...], preferred_element_type=jnp.float32) + bk[...]
    v_out[...] = jnp.dot(x, wv[...], preferred_element_type=jnp.float32) + bv[...]


def _proj(x, wq, bq, wk, bk, wv, bv):
    grid = (N_NODES // _PROJ_TILE,)
    wspec = pl.BlockSpec((D, D), lambda i: (0, 0))
    bspec = pl.BlockSpec((1, D), lambda i: (0, 0))
    rspec = pl.BlockSpec((_PROJ_TILE, D), lambda i: (i, 0))
    return pl.pallas_call(
        _proj_body,
        grid=grid,
        in_specs=[rspec, wspec, bspec, wspec, bspec, wspec, bspec],
        out_specs=[rspec, rspec, rspec],
        out_shape=[jax.ShapeDtypeStruct((N_NODES, D), jnp.float32)] * 3,
    )(x, wq, bq, wk, bk, wv, bv)


_EPW = E_EDGES // NW
_GCH = 64
_GNCH = _EPW // _GCH
_GPAIR = (_GNCH - 2) // 2
_GTAIL = _EPW - _GNCH * _GCH


def _gather_body(ktab, qtab, vtab, src, dst, ks_out, qd_out, vs_out,
                 i0s, i0d, b0k, b0q, b0v, i1s, i1d, b1k, b1q, b1v,
                 its, itd, btk, btq, btv,
                 g0, g1, w0, w1, gt):
    wid = lax.axis_index("s") * NC + lax.axis_index("c")
    base = wid * _EPW

    sets = ((i0s, i0d, b0k, b0q, b0v, g0, w0),
            (i1s, i1d, b1k, b1q, b1v, g1, w1))

    def rd(b, off):
        isv, idv, bk, bq, bv, g, _ = sets[b]
        pltpu.sync_copy(src.at[pl.ds(off, _GCH)], isv)
        pltpu.sync_copy(dst.at[pl.ds(off, _GCH)], idv)
        pltpu.async_copy(ktab.at[isv], bk, g)
        pltpu.async_copy(qtab.at[idv], bq, g)
        pltpu.async_copy(vtab.at[isv], bv, g)

    def dgwr(b, off):
        isv, idv, bk, bq, bv, g, w = sets[b]
        pltpu.make_async_copy(ktab.at[isv], bk, g).wait()
        pltpu.make_async_copy(qtab.at[idv], bq, g).wait()
        pltpu.make_async_copy(vtab.at[isv], bv, g).wait()
        pltpu.async_copy(bk, ks_out.at[pl.ds(off, _GCH)], w)
        pltpu.async_copy(bq, qd_out.at[pl.ds(off, _GCH)], w)
        pltpu.async_copy(bv, vs_out.at[pl.ds(off, _GCH)], w)

    def ww(b, off):
        isv, idv, bk, bq, bv, g, w = sets[b]
        pltpu.make_async_copy(bk, ks_out.at[pl.ds(off, _GCH)], w).wait()
        pltpu.make_async_copy(bq, qd_out.at[pl.ds(off, _GCH)], w).wait()
        pltpu.make_async_copy(bv, vs_out.at[pl.ds(off, _GCH)], w).wait()

    rd(0, base)

    def body(i, carry):
        c0 = base + (2 * i) * _GCH
        c1 = c0 + _GCH
        c2 = c1 + _GCH
        rd(1, c1)
        dgwr(0, c0)
        ww(0, c0)
        rd(0, c2)
        dgwr(1, c1)
        ww(1, c1)
        return carry

    lax.fori_loop(0, _GPAIR, body, 0)

    c76 = base + (_GNCH - 2) * _GCH
    c77 = c76 + _GCH
    dgwr(0, c76)
    rd(1, c77)
    dgwr(1, c77)
    ww(0, c76)
    ww(1, c77)

    offt = base + _GNCH * _GCH
    pltpu.sync_copy(src.at[pl.ds(offt, _GTAIL)], its)
    pltpu.sync_copy(dst.at[pl.ds(offt, _GTAIL)], itd)
    ck = pltpu.async_copy(ktab.at[its], btk, gt)
    cq = pltpu.async_copy(qtab.at[itd], btq, gt)
    cv = pltpu.async_copy(vtab.at[its], btv, gt)
    ck.wait()
    cq.wait()
    cv.wait()
    pltpu.sync_copy(btk, ks_out.at[pl.ds(offt, _GTAIL)])
    pltpu.sync_copy(btq, qd_out.at[pl.ds(offt, _GTAIL)])
    pltpu.sync_copy(btv, vs_out.at[pl.ds(offt, _GTAIL)])


_gather = functools.partial(
    pl.kernel,
    mesh=plsc.VectorSubcoreMesh(core_axis_name="c", subcore_axis_name="s",
                                num_cores=NC, num_subcores=NS),
    out_type=[jax.ShapeDtypeStruct((E_EDGES, D), jnp.float32)] * 3,
    scratch_types=(
        [pltpu.VMEM((_GCH,), jnp.int32)] * 2
        + [pltpu.VMEM((_GCH, D), jnp.float32)] * 3
        + [pltpu.VMEM((_GCH,), jnp.int32)] * 2
        + [pltpu.VMEM((_GCH, D), jnp.float32)] * 3
        + [pltpu.VMEM((_GTAIL,), jnp.int32)] * 2
        + [pltpu.VMEM((_GTAIL, D), jnp.float32)] * 3
        + [pltpu.SemaphoreType.DMA] * 5
    ),
)(_gather_body)


_ETILE = 2000


def _edge_body(ea_ref, ks_ref, qd_ref, vs_ref, dst_ref, we_ref, be_ref,
               msg3_out, scp_out):
    eh = jnp.dot(ea_ref[...], we_ref[...], preferred_element_type=jnp.float32) + be_ref[...]
    pe = ks_ref[...] * qd_ref[...] * eh
    d_i = lax.broadcasted_iota(jnp.int32, (D, H), 0)
    h_i = lax.broadcasted_iota(jnp.int32, (D, H), 1)
    seg = jnp.where(d_i // DH == h_i, SCALE, 0.0)
    raw = jnp.dot(pe, seg, preferred_element_type=jnp.float32)
    sc = jnp.exp(jnp.clip(raw, -5.0, 5.0))
    d_i2 = lax.broadcasted_iota(jnp.int32, (H, D), 1)
    h_i2 = lax.broadcasted_iota(jnp.int32, (H, D), 0)
    rep = jnp.where(d_i2 // DH == h_i2, 1.0, 0.0)
    msg = vs_ref[...] * jnp.dot(sc, rep, preferred_element_type=jnp.float32)
    msg3_out[0] = msg[:, :D // 2]
    msg3_out[1] = msg[:, D // 2:]
    dd = dst_ref[...]
    k_i = lax.broadcasted_iota(jnp.int32, (_ETILE, 8), 1)
    oh = jnp.where(dd - (dd // 8) * 8 == k_i, 1.0, 0.0)
    k_i2 = lax.broadcasted_iota(jnp.int32, (8, _HD), 0)
    d_i3 = lax.broadcasted_iota(jnp.int32, (8, _HD), 1)
    r8 = jnp.where(d_i3 // (2 * H) == k_i2, 1.0, 0.0)
    ohrep = jnp.dot(oh, r8, preferred_element_type=jnp.float32)
    sc_pad = jnp.concatenate([sc, jnp.zeros_like(sc)], axis=1)
    sc_tile = jnp.concatenate([sc_pad] * 8, axis=1)
    scp_out[...] = sc_tile * ohrep


def _edge(edge_attr, ks, qd, vs, dst2d, we, be):
    grid = (E_EDGES // _ETILE,)
    rspec = pl.BlockSpec((_ETILE, D), lambda i: (i, 0))
    return pl.pallas_call(
        _edge_body,
        grid=grid,
        in_specs=[rspec, rspec, rspec, rspec,
                  pl.BlockSpec((_ETILE, 1), lambda i: (i, 0)),
                  pl.BlockSpec((D, D), lambda i: (0, 0)),
                  pl.BlockSpec((1, D), lambda i: (0, 0))],
        out_specs=[pl.BlockSpec((NC, _ETILE, D // 2), lambda i: (0, i, 0)),
                   pl.BlockSpec((_ETILE, _HD), lambda i: (i, 0))],
        out_shape=[jax.ShapeDtypeStruct((NC, E_EDGES, D // 2), jnp.float32),
                   jax.ShapeDtypeStruct((E_EDGES, _HD), jnp.float32)],
    )(edge_attr, ks, qd, vs, dst2d, we, be)


_EPS = E_EDGES // NS
_SCH = 64
_SNCH = _EPS // _SCH
_SPAIR = (_SNCH - 2) // 2
_STAIL = _EPS - _SNCH * _SCH
_NPAD = 10240
_RPS = _NPAD // NS
_HD = D // 2
_ZW = 2 * H
_NZ = _NPAD // 8
_ZRPS = _NZ // NS


def _scatter_body(msg3, scp, dst, dst8, wv3_out, z_out,
                  acc, zacc, i0, i0b, d0, z0, i1, i1b, d1, z1,
                  it_, it2_, idxz_v, idxz16_v, r0, r1, a0, a1, st):
    c = lax.axis_index("c")
    s = lax.axis_index("s")
    zero16 = jnp.zeros((16,), jnp.float32)

    def zbody(t, carry):
        r = t // 8
        j = t - r * 8
        d0[r, pl.ds(j * 16, 16)] = zero16
        return carry

    lax.fori_loop(0, _SCH * 8, zbody, 0)

    def set_idxz(base_r):
        for j in range(4):
            idxz_v[pl.ds(j * 16, 16)] = base_r + j * 16 + lax.iota(jnp.int32, 16)

    for kk in range(10):
        set_idxz(s * _RPS + kk * _SCH)
        pltpu.sync_copy(d0, acc.at[idxz_v])
    set_idxz(s * _ZRPS)
    pltpu.sync_copy(d0, zacc.at[idxz_v])
    idxz16_v[...] = s * _ZRPS + _SCH + lax.iota(jnp.int32, 16)
    pltpu.sync_copy(d0.at[pl.ds(0, 16)], zacc.at[idxz16_v])
    plsc.subcore_barrier()

    base = s * _EPS
    sets = ((i0, i0b, d0, z0, r0, a0), (i1, i1b, d1, z1, r1, a1))

    def rd(b, off):
        ix, ix2, dd, zz, rs, _ = sets[b]
        pltpu.async_copy(dst.at[pl.ds(off, _SCH)], ix, rs)
        pltpu.async_copy(dst8.at[pl.ds(off, _SCH)], ix2, rs)
        pltpu.async_copy(msg3.at[c, pl.ds(off, _SCH)], dd, rs)
        pltpu.async_copy(scp.at[pl.ds(off, _SCH)], zz, rs)

    def add(b, off):
        ix, ix2, dd, zz, rs, asm = sets[b]
        pltpu.make_async_copy(dst.at[pl.ds(off, _SCH)], ix, rs).wait()
        pltpu.make_async_copy(dst8.at[pl.ds(off, _SCH)], ix2, rs).wait()
        pltpu.make_async_copy(msg3.at[c, pl.ds(off, _SCH)], dd, rs).wait()
        pltpu.make_async_copy(scp.at[pl.ds(off, _SCH)], zz, rs).wait()
        pltpu.async_copy(dd, acc.at[ix], asm, add=True)
        pltpu.async_copy(zz, zacc.at[ix2], asm, add=True)

    def wa(b):
        ix, ix2, dd, zz, rs, asm = sets[b]
        pltpu.make_async_copy(dd, acc.at[ix], asm).wait()
        pltpu.make_async_copy(zz, zacc.at[ix2], asm).wait()

    rd(0, base)

    def body(i, carry):
        c0 = base + (2 * i) * _SCH
        c1 = c0 + _SCH
        c2 = c1 + _SCH
        rd(1, c1)
        add(0, c0)
        wa(0)
        rd(0, c2)
        add(1, c1)
        wa(1)
        return carry

    lax.fori_loop(0, _SPAIR, body, 0)

    cA = base + (_SNCH - 2) * _SCH
    cB = cA + _SCH
    add(0, cA)
    rd(1, cB)
    add(1, cB)
    wa(0)
    wa(1)

    offt = base + _SNCH * _SCH
    t1 = pltpu.async_copy(dst.at[pl.ds(offt, _STAIL)], it_, st)
    t2 = pltpu.async_copy(dst8.at[pl.ds(offt, _STAIL)], it2_, st)
    t3 = pltpu.async_copy(msg3.at[c, pl.ds(offt, _STAIL)], d0.at[pl.ds(0, _STAIL)], st)
    t4 = pltpu.async_copy(scp.at[pl.ds(offt, _STAIL)], z0.at[pl.ds(0, _STAIL)], st)
    t1.wait()
    t2.wait()
    t3.wait()
    t4.wait()
    ta = pltpu.async_copy(d0.at[pl.ds(0, _STAIL)], acc.at[it_], st, add=True)
    tb = pltpu.async_copy(z0.at[pl.ds(0, _STAIL)], zacc.at[it2_], st, add=True)
    ta.wait()
    tb.wait()
    plsc.subcore_barrier()

    for kk in range(10):
        row = s * _RPS + kk * _SCH
        set_idxz(row)
        pltpu.async_copy(acc.at[idxz_v], d0, st).wait()
        pltpu.sync_copy(d0, wv3_out.at[c, pl.ds(row, _SCH)])
    set_idxz(s * _ZRPS)
    pltpu.async_copy(zacc.at[idxz_v], d0, st).wait()
    pltpu.sync_copy(d0, z_out.at[pl.ds(s * _ZRPS, _SCH)])
    pltpu.async_copy(zacc.at[idxz16_v], d0.at[pl.ds(0, 16)], st).wait()
    pltpu.sync_copy(d0.at[pl.ds(0, 16)], z_out.at[pl.ds(s * _ZRPS + _SCH, 16)])


_scatter = functools.partial(
    pl.kernel,
    mesh=plsc.VectorSubcoreMesh(core_axis_name="c", subcore_axis_name="s",
                                num_cores=NC, num_subcores=NS),
    out_type=[jax.ShapeDtypeStruct((NC, _NPAD, _HD), jnp.float32),
              jax.ShapeDtypeStruct((_NZ, _HD), jnp.float32)],
    scratch_types=(
        [pltpu.VMEM_SHARED((_NPAD, _HD), jnp.float32),
         pltpu.VMEM_SHARED((_NZ, _HD), jnp.float32)]
        + [pltpu.VMEM((_SCH,), jnp.int32), pltpu.VMEM((_SCH,), jnp.int32),
           pltpu.VMEM((_SCH, _HD), jnp.float32), pltpu.VMEM((_SCH, _HD), jnp.float32)] * 2
        + [pltpu.VMEM((_STAIL,), jnp.int32), pltpu.VMEM((_STAIL,), jnp.int32),
           pltpu.VMEM((_SCH,), jnp.int32), pltpu.VMEM((16,), jnp.int32)]
        + [pltpu.SemaphoreType.DMA] * 5
    ),
)(_scatter_body)


_FTILE = 1000


def _fin_body(wva_ref, wvb_ref, z_ref, out_ref):
    h_i = lax.broadcasted_iota(jnp.int32, (_ZW, _HD), 0)
    d_i = lax.broadcasted_iota(jnp.int32, (_ZW, _HD), 1)
    rep_a = jnp.where(h_i == d_i // DH, 1.0, 0.0)
    rep_b = jnp.where(h_i == d_i // DH + H // 2, 1.0, 0.0)
    za = jnp.dot(z_ref[...], rep_a, preferred_element_type=jnp.float32)
    zb = jnp.dot(z_ref[...], rep_b, preferred_element_type=jnp.float32)
    out_ref[:, :_HD] = wva_ref[...] / (za + 1e-6)
    out_ref[:, _HD:] = wvb_ref[...] / (zb + 1e-6)


def _fin(wva, wvb, z):
    grid = (N_NODES // _FTILE,)
    return pl.pallas_call(
        _fin_body,
        grid=grid,
        in_specs=[pl.BlockSpec((_FTILE, _HD), lambda i: (i, 0)),
                  pl.BlockSpec((_FTILE, _HD), lambda i: (i, 0)),
                  pl.BlockSpec((_FTILE, _ZW), lambda i: (i, 0))],
        out_specs=pl.BlockSpec((_FTILE, D), lambda i: (i, 0)),
        out_shape=jax.ShapeDtypeStruct((N_NODES, D), jnp.float32),
    )(wva, wvb, z)


def kernel(x, edge_attr, Wq, bq, Wk, bk, We, be, Wv, bv, edge_index):
    src = edge_index[0]
    dst = edge_index[1]
    q, k, v = _proj(x, Wq, bq.reshape(1, D), Wk, bk.reshape(1, D),
                    Wv, bv.reshape(1, D))
    ks, qd, vs = _gather(k, q, v, src, dst)
    msg3, scp = _edge(edge_attr, ks, qd, vs, dst.reshape(E_EDGES, 1),
                      We, be.reshape(1, D))
    wv3, z2 = _scatter(msg3, scp, dst, dst // 8)
    return _fin(wv3[0], wv3[1], z2.reshape(_NPAD, _ZW))
